# Initial kernel scaffold; baseline (speedup 1.0000x reference)
import functools

import jax
import jax.numpy as jnp
from jax import lax
from jax.experimental import pallas as pl
from jax.experimental.pallas import tpu as pltpu
from jax.experimental.pallas import tpu_sc as plsc

N_STEPS = 3
H = 64
CHUNK = 125


def _make_sc_scatter(n_nodes, n_edges):
    info = plsc.get_sparse_core_info()
    nc, ns = info.num_cores, info.num_subcores
    nw = nc * ns
    assert n_edges % (nw * CHUNK) == 0
    rows_per_w = n_edges // (nw * CHUNK)
    assert n_nodes % ns == 0
    npt = n_nodes // ns

    mesh = plsc.VectorSubcoreMesh(core_axis_name="c", subcore_axis_name="s")

    @functools.partial(
        pl.kernel,
        mesh=mesh,
        out_type=jax.ShapeDtypeStruct((nc * n_nodes, H), jnp.float32),
        scratch_types=[
            pltpu.VMEM((rows_per_w, CHUNK), jnp.int32),
            pltpu.VMEM((rows_per_w, CHUNK), jnp.int32),
            pltpu.VMEM((CHUNK, H), jnp.float32),
            pltpu.VMEM((npt, H), jnp.float32),
            pltpu.VMEM_SHARED((n_nodes, H), jnp.float32),
            pltpu.SemaphoreType.DMA,
        ],
    )
    def sc_scatter(table_hbm, src_hbm, dst_hbm, zeros_hbm, out_hbm,
                   src_v, dst_v, rows_v, wb_v, accum_sh, sem):
        c = lax.axis_index("c")
        s = lax.axis_index("s")
        wid = s * nc + c
        row0 = wid * rows_per_w
        pltpu.sync_copy(src_hbm.at[pl.ds(row0, rows_per_w)], src_v)
        pltpu.sync_copy(dst_hbm.at[pl.ds(row0, rows_per_w)], dst_v)

        zrow0 = s * npt
        pltpu.sync_copy(zeros_hbm.at[pl.ds(zrow0, npt)], wb_v)
        pltpu.sync_copy(wb_v, accum_sh.at[pl.ds(zrow0, npt)])
        plsc.subcore_barrier()

        def body(j, carry):
            pltpu.async_copy(table_hbm.at[src_v.at[j]], rows_v, sem).wait()
            pltpu.sync_copy(rows_v, accum_sh.at[dst_v.at[j]], add=True)
            return carry

        lax.fori_loop(0, rows_per_w, body, 0)

        plsc.subcore_barrier()
        pltpu.sync_copy(accum_sh.at[pl.ds(zrow0, npt)], wb_v)
        pltpu.sync_copy(wb_v, out_hbm.at[pl.ds(c * n_nodes + zrow0, npt)])

    return sc_scatter


def _dg(a, w):
    return lax.dot_general(a, w, (((1,), (1,)), ((), ())),
                           preferred_element_type=jnp.float32)


def _enc_body(x_ref, we_ref, be_ref, wm_ref, bm_ref, h_ref, hw_ref):
    h = _dg(x_ref[...], we_ref[...]) + be_ref[...]
    h_ref[...] = h
    hw_ref[...] = _dg(h, wm_ref[...]) + bm_ref[...]


def _gru_body(p0_ref, p1_ref, h_ref, wir_ref, wiz_ref, win_ref,
              whr_ref, whz_ref, whn_ref, br_ref, bz_ref, bin_ref, bhn_ref,
              wm_ref, bm_ref, hn_ref, hw_ref):
    m = p0_ref[...] + p1_ref[...]
    h = h_ref[...]
    r = jax.nn.sigmoid(_dg(m, wir_ref[...]) + _dg(h, whr_ref[...]) + br_ref[...])
    z = jax.nn.sigmoid(_dg(m, wiz_ref[...]) + _dg(h, whz_ref[...]) + bz_ref[...])
    n = jnp.tanh(_dg(m, win_ref[...]) + bin_ref[...]
                 + r * (_dg(h, whn_ref[...]) + bhn_ref[...]))
    hn = (1.0 - z) * n + z * h
    hn_ref[...] = hn
    hw_ref[...] = _dg(hn, wm_ref[...]) + bm_ref[...]


def _readout_body(h_ref, w1_ref, b1_ref, w2_ref, b2_ref, o_ref):
    g = jnp.sum(h_ref[...], axis=0, keepdims=True)
    t = jnp.maximum(_dg(g, w1_ref[...]) + b1_ref[...], 0.0)
    o_ref[...] = _dg(t, w2_ref[...]) + b2_ref[...]


def kernel(x, edge_index, W_enc, b_enc, W_msg, b_msg, W_ih, b_ih, W_hh, b_hh,
           W_r1, b_r1, W_r2, b_r2):
    n_nodes, d_feat = x.shape
    n_edges = edge_index.shape[1]
    out_dim = W_r2.shape[0]

    info = plsc.get_sparse_core_info()
    nc = info.num_cores

    ei = edge_index.astype(jnp.int32)
    srcr = ei[0].reshape(-1, CHUNK)
    dstr = ei[1].reshape(-1, CHUNK)
    zeros = jnp.zeros((n_nodes, H), jnp.float32)

    sc_scatter = _make_sc_scatter(n_nodes, n_edges)

    blk = 1000
    assert n_nodes % blk == 0
    grid = n_nodes // blk

    b_enc2 = b_enc.reshape(1, H)
    b_msg2 = b_msg.reshape(1, H)
    b_r = (b_ih[:H] + b_hh[:H]).reshape(1, H)
    b_z = (b_ih[H:2 * H] + b_hh[H:2 * H]).reshape(1, H)
    b_in = b_ih[2 * H:].reshape(1, H)
    b_hn = b_hh[2 * H:].reshape(1, H)
    W_ir, W_iz, W_in = W_ih[:H], W_ih[H:2 * H], W_ih[2 * H:]
    W_hr, W_hz, W_hn = W_hh[:H], W_hh[H:2 * H], W_hh[2 * H:]

    full = lambda shape: pl.BlockSpec(shape, lambda i: (0, 0))
    rows = lambda w: pl.BlockSpec((blk, w), lambda i: (i, 0))

    h, hw = pl.pallas_call(
        _enc_body,
        grid=(grid,),
        in_specs=[rows(d_feat), full((H, d_feat)), full((1, H)),
                  full((H, H)), full((1, H))],
        out_specs=[rows(H), rows(H)],
        out_shape=[jax.ShapeDtypeStruct((n_nodes, H), jnp.float32)] * 2,
    )(x, W_enc, b_enc2, W_msg, b_msg2)

    gru_call = pl.pallas_call(
        _gru_body,
        grid=(grid,),
        in_specs=[pl.BlockSpec((blk, H), lambda i: (i, 0)),
                  pl.BlockSpec((blk, H), lambda i: (i + grid, 0)),
                  rows(H)]
                 + [full((H, H))] * 6 + [full((1, H))] * 4
                 + [full((H, H)), full((1, H))],
        out_specs=[rows(H), rows(H)],
        out_shape=[jax.ShapeDtypeStruct((n_nodes, H), jnp.float32)] * 2,
    )

    for _ in range(N_STEPS):
        parts = sc_scatter(hw, srcr, dstr, zeros)
        h, hw = gru_call(parts, parts, h, W_ir, W_iz, W_in, W_hr, W_hz, W_hn,
                         b_r, b_z, b_in, b_hn, W_msg, b_msg2)

    out = pl.pallas_call(
        _readout_body,
        in_specs=[pl.BlockSpec((n_nodes, H), lambda: (0, 0)),
                  pl.BlockSpec((H, H), lambda: (0, 0)),
                  pl.BlockSpec((1, H), lambda: (0, 0)),
                  pl.BlockSpec((out_dim, H), lambda: (0, 0)),
                  pl.BlockSpec((1, out_dim), lambda: (0, 0))],
        out_specs=pl.BlockSpec((1, out_dim), lambda: (0, 0)),
        out_shape=jax.ShapeDtypeStruct((1, out_dim), jnp.float32),
    )(h, W_r1, b_r1.reshape(1, H), W_r2, b_r2.reshape(1, out_dim))
    return out


# speedup vs baseline: 8.4840x; 8.4840x over previous
import functools

import jax
import jax.numpy as jnp
from jax import lax
from jax.experimental import pallas as pl
from jax.experimental.pallas import tpu as pltpu
from jax.experimental.pallas import tpu_sc as plsc

N_STEPS = 3
H = 64
CHUNK = 125


def _make_sc_scatter(n_nodes, n_edges):
    info = plsc.get_sparse_core_info()
    nc, ns = info.num_cores, info.num_subcores
    nw = nc * ns
    assert n_edges % (nw * CHUNK) == 0
    rows_per_w = n_edges // (nw * CHUNK)
    npt = 640
    npt_stride = 624
    assert (ns - 1) * npt_stride + npt == n_nodes

    mesh = plsc.VectorSubcoreMesh(core_axis_name="c", subcore_axis_name="s")

    @functools.partial(
        pl.kernel,
        mesh=mesh,
        out_type=jax.ShapeDtypeStruct((nc * n_nodes, H), jnp.float32),
        scratch_types=[
            pltpu.VMEM((rows_per_w, CHUNK), jnp.int32),
            pltpu.VMEM((rows_per_w, CHUNK), jnp.int32),
            pltpu.VMEM((CHUNK, H), jnp.float32),
            pltpu.VMEM((npt, H), jnp.float32),
            pltpu.VMEM_SHARED((n_nodes, H), jnp.float32),
            pltpu.SemaphoreType.DMA,
        ],
        compiler_params=pltpu.CompilerParams(use_tc_tiling_on_sc=False),
    )
    def sc_scatter(table_hbm, src_hbm, dst_hbm, zeros_hbm, out_hbm,
                   src_v, dst_v, rows_v, wb_v, accum_sh, sem):
        c = lax.axis_index("c")
        s = lax.axis_index("s")
        wid = s * nc + c
        row0 = wid * rows_per_w
        pltpu.sync_copy(src_hbm.at[pl.ds(row0, rows_per_w)], src_v)
        pltpu.sync_copy(dst_hbm.at[pl.ds(row0, rows_per_w)], dst_v)

        zrow0 = s * npt_stride
        pltpu.sync_copy(zeros_hbm.at[pl.ds(zrow0, npt)], wb_v)
        pltpu.sync_copy(wb_v, accum_sh.at[pl.ds(zrow0, npt)])
        plsc.subcore_barrier()

        def body(j, carry):
            pltpu.async_copy(table_hbm.at[src_v.at[j]], rows_v, sem).wait()
            pltpu.sync_copy(rows_v, accum_sh.at[dst_v.at[j]], add=True)
            return carry

        lax.fori_loop(0, rows_per_w, body, 0)

        plsc.subcore_barrier()
        pltpu.sync_copy(accum_sh.at[pl.ds(zrow0, npt)], wb_v)
        pltpu.sync_copy(wb_v, out_hbm.at[pl.ds(c * n_nodes + zrow0, npt)])

    return sc_scatter


def _dg(a, w):
    return lax.dot_general(a, w, (((1,), (1,)), ((), ())),
                           preferred_element_type=jnp.float32)


def _enc_body(x_ref, we_ref, be_ref, wm_ref, bm_ref, h_ref, hw_ref):
    h = _dg(x_ref[...], we_ref[...]) + be_ref[...]
    h_ref[...] = h
    hw_ref[...] = _dg(h, wm_ref[...]) + bm_ref[...]


def _gru_body(p0_ref, p1_ref, h_ref, wir_ref, wiz_ref, win_ref,
              whr_ref, whz_ref, whn_ref, br_ref, bz_ref, bin_ref, bhn_ref,
              wm_ref, bm_ref, hn_ref, hw_ref):
    m = p0_ref[...] + p1_ref[...]
    h = h_ref[...]
    r = jax.nn.sigmoid(_dg(m, wir_ref[...]) + _dg(h, whr_ref[...]) + br_ref[...])
    z = jax.nn.sigmoid(_dg(m, wiz_ref[...]) + _dg(h, whz_ref[...]) + bz_ref[...])
    n = jnp.tanh(_dg(m, win_ref[...]) + bin_ref[...]
                 + r * (_dg(h, whn_ref[...]) + bhn_ref[...]))
    hn = (1.0 - z) * n + z * h
    hn_ref[...] = hn
    hw_ref[...] = _dg(hn, wm_ref[...]) + bm_ref[...]


def _readout_body(h_ref, w1_ref, b1_ref, w2_ref, b2_ref, o_ref):
    g = jnp.sum(h_ref[...], axis=0, keepdims=True)
    t = jnp.maximum(_dg(g, w1_ref[...]) + b1_ref[...], 0.0)
    o_ref[...] = _dg(t, w2_ref[...]) + b2_ref[...]


def kernel(x, edge_index, W_enc, b_enc, W_msg, b_msg, W_ih, b_ih, W_hh, b_hh,
           W_r1, b_r1, W_r2, b_r2):
    n_nodes, d_feat = x.shape
    n_edges = edge_index.shape[1]
    out_dim = W_r2.shape[0]

    info = plsc.get_sparse_core_info()
    nc = info.num_cores

    ei = edge_index.astype(jnp.int32)
    srcr = ei[0].reshape(-1, CHUNK)
    dstr = ei[1].reshape(-1, CHUNK)
    zeros = jnp.zeros((n_nodes, H), jnp.float32)

    sc_scatter = _make_sc_scatter(n_nodes, n_edges)

    blk = 1000
    assert n_nodes % blk == 0
    grid = n_nodes // blk

    b_enc2 = b_enc.reshape(1, H)
    b_msg2 = b_msg.reshape(1, H)
    b_r = (b_ih[:H] + b_hh[:H]).reshape(1, H)
    b_z = (b_ih[H:2 * H] + b_hh[H:2 * H]).reshape(1, H)
    b_in = b_ih[2 * H:].reshape(1, H)
    b_hn = b_hh[2 * H:].reshape(1, H)
    W_ir, W_iz, W_in = W_ih[:H], W_ih[H:2 * H], W_ih[2 * H:]
    W_hr, W_hz, W_hn = W_hh[:H], W_hh[H:2 * H], W_hh[2 * H:]

    full = lambda shape: pl.BlockSpec(shape, lambda i: (0, 0))
    rows = lambda w: pl.BlockSpec((blk, w), lambda i: (i, 0))

    h, hw = pl.pallas_call(
        _enc_body,
        grid=(grid,),
        in_specs=[rows(d_feat), full((H, d_feat)), full((1, H)),
                  full((H, H)), full((1, H))],
        out_specs=[rows(H), rows(H)],
        out_shape=[jax.ShapeDtypeStruct((n_nodes, H), jnp.float32)] * 2,
    )(x, W_enc, b_enc2, W_msg, b_msg2)

    gru_call = pl.pallas_call(
        _gru_body,
        grid=(grid,),
        in_specs=[pl.BlockSpec((blk, H), lambda i: (i, 0)),
                  pl.BlockSpec((blk, H), lambda i: (i + grid, 0)),
                  rows(H)]
                 + [full((H, H))] * 6 + [full((1, H))] * 4
                 + [full((H, H)), full((1, H))],
        out_specs=[rows(H), rows(H)],
        out_shape=[jax.ShapeDtypeStruct((n_nodes, H), jnp.float32)] * 2,
    )

    for _ in range(N_STEPS):
        parts = sc_scatter(hw, srcr, dstr, zeros)
        h, hw = gru_call(parts, parts, h, W_ir, W_iz, W_in, W_hr, W_hz, W_hn,
                         b_r, b_z, b_in, b_hn, W_msg, b_msg2)

    out = pl.pallas_call(
        _readout_body,
        in_specs=[pl.BlockSpec((n_nodes, H), lambda: (0, 0)),
                  pl.BlockSpec((H, H), lambda: (0, 0)),
                  pl.BlockSpec((1, H), lambda: (0, 0)),
                  pl.BlockSpec((out_dim, H), lambda: (0, 0)),
                  pl.BlockSpec((1, out_dim), lambda: (0, 0))],
        out_specs=pl.BlockSpec((1, out_dim), lambda: (0, 0)),
        out_shape=jax.ShapeDtypeStruct((1, out_dim), jnp.float32),
    )(h, W_r1, b_r1.reshape(1, H), W_r2, b_r2.reshape(1, out_dim))
    return out


# speedup vs baseline: 11.6093x; 1.3684x over previous
import functools

import jax
import jax.numpy as jnp
from jax import lax
from jax.experimental import pallas as pl
from jax.experimental.pallas import tpu as pltpu
from jax.experimental.pallas import tpu_sc as plsc

N_STEPS = 3
H = 64
CHUNK = 125


def _make_sc_scatter(n_nodes, n_edges):
    info = plsc.get_sparse_core_info()
    nc, ns = info.num_cores, info.num_subcores
    nw = nc * ns
    assert n_edges % (nw * CHUNK) == 0
    rows_per_w = n_edges // (nw * CHUNK)
    npt = 640
    npt_stride = 624
    assert (ns - 1) * npt_stride + npt == n_nodes

    mesh = plsc.VectorSubcoreMesh(core_axis_name="c", subcore_axis_name="s")

    nbuf = 4
    assert rows_per_w % nbuf == 0
    n_groups = rows_per_w // nbuf

    @functools.partial(
        pl.kernel,
        mesh=mesh,
        out_type=jax.ShapeDtypeStruct((nc * n_nodes, H), jnp.float32),
        scratch_types=[
            pltpu.VMEM((rows_per_w, CHUNK), jnp.int32),
            pltpu.VMEM((rows_per_w, CHUNK), jnp.int32),
            pltpu.VMEM((nbuf, CHUNK, H), jnp.float32),
            pltpu.VMEM((2, 64, H), jnp.float32),
            pltpu.VMEM_SHARED((n_nodes, H), jnp.float32),
            pltpu.SemaphoreType.DMA((nbuf,)),
            pltpu.SemaphoreType.DMA((nbuf,)),
            pltpu.SemaphoreType.DMA((2,)),
            pltpu.SemaphoreType.DMA,
        ],
        compiler_params=pltpu.CompilerParams(use_tc_tiling_on_sc=False),
    )
    def sc_scatter(table_hbm, src_hbm, dst_hbm, out_hbm,
                   src_v, dst_v, rows_v, wb_v, accum_sh, gsem, ssem, isem, zsem):
        c = lax.axis_index("c")
        s = lax.axis_index("s")
        wid = s * nc + c
        row0 = wid * rows_per_w
        i1 = pltpu.async_copy(src_hbm.at[pl.ds(row0, rows_per_w)], src_v,
                              isem.at[0])
        i2 = pltpu.async_copy(dst_hbm.at[pl.ds(row0, rows_per_w)], dst_v,
                              isem.at[1])

        zvec = jnp.zeros((16,), jnp.float32)

        def zrow(i, carry):
            for k in range(H // 16):
                wb_v[0, i, pl.ds(k * 16, 16)] = zvec
            return carry

        lax.fori_loop(0, 64, zrow, 0)
        zrow0 = s * npt_stride
        zs = [pltpu.async_copy(wb_v.at[0],
                               accum_sh.at[pl.ds(zrow0 + 64 * j, 64)], zsem)
              for j in range(npt // 64)]
        for z in zs:
            z.wait()
        i1.wait()
        i2.wait()
        plsc.subcore_barrier()

        def group(g, carry):
            j0 = g * nbuf
            gs = [pltpu.async_copy(table_hbm.at[src_v.at[j0 + b]],
                                   rows_v.at[b], gsem.at[b])
                  for b in range(nbuf)]
            ss = []
            for b in range(nbuf):
                gs[b].wait()
                ss.append(pltpu.async_copy(rows_v.at[b],
                                           accum_sh.at[dst_v.at[j0 + b]],
                                           ssem.at[b], add=True))
            for b in range(nbuf):
                ss[b].wait()
            return carry

        lax.fori_loop(0, n_groups, group, 0)

        plsc.subcore_barrier()
        orow0 = c * n_nodes + zrow0
        wbs = []
        for j in range(npt // 64):
            b = j % 2
            if j >= 2:
                wbs[j - 2].wait()
            pltpu.sync_copy(accum_sh.at[pl.ds(zrow0 + 64 * j, 64)], wb_v.at[b])
            wbs.append(pltpu.async_copy(
                wb_v.at[b], out_hbm.at[pl.ds(orow0 + 64 * j, 64)], isem.at[b]))
        wbs[-2].wait()
        wbs[-1].wait()

    return sc_scatter


def _dg(a, w):
    return lax.dot_general(a, w, (((1,), (1,)), ((), ())),
                           preferred_element_type=jnp.float32)


def _enc_body(x_ref, we_ref, be_ref, wm_ref, bm_ref, h_ref, hw_ref):
    h = _dg(x_ref[...], we_ref[...]) + be_ref[...]
    h_ref[...] = h
    hw_ref[...] = _dg(h, wm_ref[...]) + bm_ref[...]


def _gru_body(p0_ref, p1_ref, h_ref, wir_ref, wiz_ref, win_ref,
              whr_ref, whz_ref, whn_ref, br_ref, bz_ref, bin_ref, bhn_ref,
              wm_ref, bm_ref, hn_ref, hw_ref):
    m = p0_ref[...] + p1_ref[...]
    h = h_ref[...]
    r = jax.nn.sigmoid(_dg(m, wir_ref[...]) + _dg(h, whr_ref[...]) + br_ref[...])
    z = jax.nn.sigmoid(_dg(m, wiz_ref[...]) + _dg(h, whz_ref[...]) + bz_ref[...])
    n = jnp.tanh(_dg(m, win_ref[...]) + bin_ref[...]
                 + r * (_dg(h, whn_ref[...]) + bhn_ref[...]))
    hn = (1.0 - z) * n + z * h
    hn_ref[...] = hn
    hw_ref[...] = _dg(hn, wm_ref[...]) + bm_ref[...]


def _readout_body(h_ref, w1_ref, b1_ref, w2_ref, b2_ref, o_ref):
    g = jnp.sum(h_ref[...], axis=0, keepdims=True)
    t = jnp.maximum(_dg(g, w1_ref[...]) + b1_ref[...], 0.0)
    o_ref[...] = _dg(t, w2_ref[...]) + b2_ref[...]


def kernel(x, edge_index, W_enc, b_enc, W_msg, b_msg, W_ih, b_ih, W_hh, b_hh,
           W_r1, b_r1, W_r2, b_r2):
    n_nodes, d_feat = x.shape
    n_edges = edge_index.shape[1]
    out_dim = W_r2.shape[0]

    info = plsc.get_sparse_core_info()
    nc = info.num_cores

    ei = edge_index.astype(jnp.int32)
    srcr = ei[0].reshape(-1, CHUNK)
    dstr = ei[1].reshape(-1, CHUNK)

    sc_scatter = _make_sc_scatter(n_nodes, n_edges)

    blk = 1000
    assert n_nodes % blk == 0
    grid = n_nodes // blk

    b_enc2 = b_enc.reshape(1, H)
    b_msg2 = b_msg.reshape(1, H)
    b_r = (b_ih[:H] + b_hh[:H]).reshape(1, H)
    b_z = (b_ih[H:2 * H] + b_hh[H:2 * H]).reshape(1, H)
    b_in = b_ih[2 * H:].reshape(1, H)
    b_hn = b_hh[2 * H:].reshape(1, H)
    W_ir, W_iz, W_in = W_ih[:H], W_ih[H:2 * H], W_ih[2 * H:]
    W_hr, W_hz, W_hn = W_hh[:H], W_hh[H:2 * H], W_hh[2 * H:]

    full = lambda shape: pl.BlockSpec(shape, lambda i: (0, 0))
    rows = lambda w: pl.BlockSpec((blk, w), lambda i: (i, 0))

    h, hw = pl.pallas_call(
        _enc_body,
        grid=(grid,),
        in_specs=[rows(d_feat), full((H, d_feat)), full((1, H)),
                  full((H, H)), full((1, H))],
        out_specs=[rows(H), rows(H)],
        out_shape=[jax.ShapeDtypeStruct((n_nodes, H), jnp.float32)] * 2,
    )(x, W_enc, b_enc2, W_msg, b_msg2)

    gru_call = pl.pallas_call(
        _gru_body,
        grid=(grid,),
        in_specs=[pl.BlockSpec((blk, H), lambda i: (i, 0)),
                  pl.BlockSpec((blk, H), lambda i: (i + grid, 0)),
                  rows(H)]
                 + [full((H, H))] * 6 + [full((1, H))] * 4
                 + [full((H, H)), full((1, H))],
        out_specs=[rows(H), rows(H)],
        out_shape=[jax.ShapeDtypeStruct((n_nodes, H), jnp.float32)] * 2,
    )

    for _ in range(N_STEPS):
        parts = sc_scatter(hw, srcr, dstr)
        h, hw = gru_call(parts, parts, h, W_ir, W_iz, W_in, W_hr, W_hz, W_hn,
                         b_r, b_z, b_in, b_hn, W_msg, b_msg2)

    out = pl.pallas_call(
        _readout_body,
        in_specs=[pl.BlockSpec((n_nodes, H), lambda: (0, 0)),
                  pl.BlockSpec((H, H), lambda: (0, 0)),
                  pl.BlockSpec((1, H), lambda: (0, 0)),
                  pl.BlockSpec((out_dim, H), lambda: (0, 0)),
                  pl.BlockSpec((1, out_dim), lambda: (0, 0))],
        out_specs=pl.BlockSpec((1, out_dim), lambda: (0, 0)),
        out_shape=jax.ShapeDtypeStruct((1, out_dim), jnp.float32),
    )(h, W_r1, b_r1.reshape(1, H), W_r2, b_r2.reshape(1, out_dim))
    return out


# speedup vs baseline: 13.8797x; 1.1956x over previous
import functools

import jax
import jax.numpy as jnp
from jax import lax
from jax.experimental import pallas as pl
from jax.experimental.pallas import tpu as pltpu
from jax.experimental.pallas import tpu_sc as plsc

N_STEPS = 3
H = 64
CHUNK = 125


def _make_sc_scatter(n_nodes, n_edges):
    info = plsc.get_sparse_core_info()
    nc, ns = info.num_cores, info.num_subcores
    nw = nc * ns
    assert n_edges % (nw * CHUNK) == 0
    rows_per_w = n_edges // (nw * CHUNK)
    npt = 640
    npt_stride = 624
    assert (ns - 1) * npt_stride + npt == n_nodes

    mesh = plsc.VectorSubcoreMesh(core_axis_name="c", subcore_axis_name="s")

    nbuf = 4
    assert rows_per_w % nbuf == 0
    n_groups = rows_per_w // nbuf

    @functools.partial(
        pl.kernel,
        mesh=mesh,
        out_type=jax.ShapeDtypeStruct((nc * n_nodes, H), jnp.float32),
        scratch_types=[
            pltpu.VMEM((rows_per_w, CHUNK), jnp.int32),
            pltpu.VMEM((rows_per_w, CHUNK), jnp.int32),
            pltpu.VMEM((nbuf, CHUNK, H), jnp.float32),
            pltpu.VMEM((2, 64, H), jnp.float32),
            pltpu.VMEM_SHARED((n_nodes, H), jnp.float32),
            pltpu.SemaphoreType.DMA((nbuf,)),
            pltpu.SemaphoreType.DMA((nbuf,)),
            pltpu.SemaphoreType.DMA((2,)),
            pltpu.SemaphoreType.DMA,
        ],
        compiler_params=pltpu.CompilerParams(use_tc_tiling_on_sc=False),
    )
    def sc_scatter(table_hbm, src_hbm, dst_hbm, out_hbm,
                   src_v, dst_v, rows_v, wb_v, accum_sh, gsem, ssem, isem, zsem):
        c = lax.axis_index("c")
        s = lax.axis_index("s")
        wid = s * nc + c
        row0 = wid * rows_per_w
        i1 = pltpu.async_copy(src_hbm.at[pl.ds(row0, rows_per_w)], src_v,
                              isem.at[0])
        i2 = pltpu.async_copy(dst_hbm.at[pl.ds(row0, rows_per_w)], dst_v,
                              isem.at[1])

        zvec = jnp.zeros((16,), jnp.float32)

        def zrow(i, carry):
            for k in range(H // 16):
                wb_v[0, i, pl.ds(k * 16, 16)] = zvec
            return carry

        lax.fori_loop(0, 64, zrow, 0)
        zrow0 = s * npt_stride
        zs = [pltpu.async_copy(wb_v.at[0],
                               accum_sh.at[pl.ds(zrow0 + 64 * j, 64)], zsem)
              for j in range(npt // 64)]
        for z in zs:
            z.wait()
        i1.wait()
        i2.wait()
        plsc.subcore_barrier()

        def start_gather(j, b):
            pltpu.async_copy(table_hbm.at[src_v.at[j]], rows_v.at[b],
                             gsem.at[b])

        def wait_gather(j, b):
            pltpu.make_async_copy(table_hbm.at[src_v.at[j]], rows_v.at[b],
                                  gsem.at[b]).wait()

        def start_scatter(j, b):
            pltpu.async_copy(rows_v.at[b], accum_sh.at[dst_v.at[j]],
                             ssem.at[b], add=True)

        def wait_scatter(j, b):
            pltpu.make_async_copy(rows_v.at[b], accum_sh.at[dst_v.at[j]],
                                  ssem.at[b]).wait()

        def chunk_iter(j, b, first_group, last_group):
            b2 = (b + 2) % nbuf
            if not first_group or b >= 2:
                wait_scatter(j - 2, b2)
            if not last_group or b < 2:
                start_gather(j + 2, b2)
            wait_gather(j, b)
            start_scatter(j, b)

        start_gather(0, 0)
        start_gather(1, 1)
        for b in range(nbuf):
            chunk_iter(b, b, True, False)

        def group(g, carry):
            for b in range(nbuf):
                chunk_iter(g * nbuf + b, b, False, False)
            return carry

        lax.fori_loop(1, n_groups - 1, group, 0)
        for b in range(nbuf):
            chunk_iter((n_groups - 1) * nbuf + b, b, False, True)
        wait_scatter(rows_per_w - 2, 2)
        wait_scatter(rows_per_w - 1, 3)

        plsc.subcore_barrier()
        orow0 = c * n_nodes + zrow0
        wbs = []
        for j in range(npt // 64):
            b = j % 2
            if j >= 2:
                wbs[j - 2].wait()
            pltpu.sync_copy(accum_sh.at[pl.ds(zrow0 + 64 * j, 64)], wb_v.at[b])
            wbs.append(pltpu.async_copy(
                wb_v.at[b], out_hbm.at[pl.ds(orow0 + 64 * j, 64)], isem.at[b]))
        wbs[-2].wait()
        wbs[-1].wait()

    return sc_scatter


def _dg(a, w):
    return lax.dot_general(a, w, (((1,), (1,)), ((), ())),
                           preferred_element_type=jnp.float32)


def _enc_body(x_ref, we_ref, be_ref, wm_ref, bm_ref, h_ref, hw_ref):
    h = _dg(x_ref[...], we_ref[...]) + be_ref[...]
    h_ref[...] = h
    hw_ref[...] = _dg(h, wm_ref[...]) + bm_ref[...]


def _gru_body(p0_ref, p1_ref, h_ref, wir_ref, wiz_ref, win_ref,
              whr_ref, whz_ref, whn_ref, br_ref, bz_ref, bin_ref, bhn_ref,
              wm_ref, bm_ref, hn_ref, hw_ref):
    m = p0_ref[...] + p1_ref[...]
    h = h_ref[...]
    r = jax.nn.sigmoid(_dg(m, wir_ref[...]) + _dg(h, whr_ref[...]) + br_ref[...])
    z = jax.nn.sigmoid(_dg(m, wiz_ref[...]) + _dg(h, whz_ref[...]) + bz_ref[...])
    n = jnp.tanh(_dg(m, win_ref[...]) + bin_ref[...]
                 + r * (_dg(h, whn_ref[...]) + bhn_ref[...]))
    hn = (1.0 - z) * n + z * h
    hn_ref[...] = hn
    hw_ref[...] = _dg(hn, wm_ref[...]) + bm_ref[...]


def _readout_body(h_ref, w1_ref, b1_ref, w2_ref, b2_ref, o_ref):
    g = jnp.sum(h_ref[...], axis=0, keepdims=True)
    t = jnp.maximum(_dg(g, w1_ref[...]) + b1_ref[...], 0.0)
    o_ref[...] = _dg(t, w2_ref[...]) + b2_ref[...]


def kernel(x, edge_index, W_enc, b_enc, W_msg, b_msg, W_ih, b_ih, W_hh, b_hh,
           W_r1, b_r1, W_r2, b_r2):
    n_nodes, d_feat = x.shape
    n_edges = edge_index.shape[1]
    out_dim = W_r2.shape[0]

    info = plsc.get_sparse_core_info()
    nc = info.num_cores

    ei = edge_index.astype(jnp.int32)
    srcr = ei[0].reshape(-1, CHUNK)
    dstr = ei[1].reshape(-1, CHUNK)

    sc_scatter = _make_sc_scatter(n_nodes, n_edges)

    blk = 1000
    assert n_nodes % blk == 0
    grid = n_nodes // blk

    b_enc2 = b_enc.reshape(1, H)
    b_msg2 = b_msg.reshape(1, H)
    b_r = (b_ih[:H] + b_hh[:H]).reshape(1, H)
    b_z = (b_ih[H:2 * H] + b_hh[H:2 * H]).reshape(1, H)
    b_in = b_ih[2 * H:].reshape(1, H)
    b_hn = b_hh[2 * H:].reshape(1, H)
    W_ir, W_iz, W_in = W_ih[:H], W_ih[H:2 * H], W_ih[2 * H:]
    W_hr, W_hz, W_hn = W_hh[:H], W_hh[H:2 * H], W_hh[2 * H:]

    full = lambda shape: pl.BlockSpec(shape, lambda i: (0, 0))
    rows = lambda w: pl.BlockSpec((blk, w), lambda i: (i, 0))

    h, hw = pl.pallas_call(
        _enc_body,
        grid=(grid,),
        in_specs=[rows(d_feat), full((H, d_feat)), full((1, H)),
                  full((H, H)), full((1, H))],
        out_specs=[rows(H), rows(H)],
        out_shape=[jax.ShapeDtypeStruct((n_nodes, H), jnp.float32)] * 2,
    )(x, W_enc, b_enc2, W_msg, b_msg2)

    gru_call = pl.pallas_call(
        _gru_body,
        grid=(grid,),
        in_specs=[pl.BlockSpec((blk, H), lambda i: (i, 0)),
                  pl.BlockSpec((blk, H), lambda i: (i + grid, 0)),
                  rows(H)]
                 + [full((H, H))] * 6 + [full((1, H))] * 4
                 + [full((H, H)), full((1, H))],
        out_specs=[rows(H), rows(H)],
        out_shape=[jax.ShapeDtypeStruct((n_nodes, H), jnp.float32)] * 2,
    )

    for _ in range(N_STEPS):
        parts = sc_scatter(hw, srcr, dstr)
        h, hw = gru_call(parts, parts, h, W_ir, W_iz, W_in, W_hr, W_hz, W_hn,
                         b_r, b_z, b_in, b_hn, W_msg, b_msg2)

    out = pl.pallas_call(
        _readout_body,
        in_specs=[pl.BlockSpec((n_nodes, H), lambda: (0, 0)),
                  pl.BlockSpec((H, H), lambda: (0, 0)),
                  pl.BlockSpec((1, H), lambda: (0, 0)),
                  pl.BlockSpec((out_dim, H), lambda: (0, 0)),
                  pl.BlockSpec((1, out_dim), lambda: (0, 0))],
        out_specs=pl.BlockSpec((1, out_dim), lambda: (0, 0)),
        out_shape=jax.ShapeDtypeStruct((1, out_dim), jnp.float32),
    )(h, W_r1, b_r1.reshape(1, H), W_r2, b_r2.reshape(1, out_dim))
    return out


# speedup vs baseline: 15.2841x; 1.1012x over previous
import functools

import jax
import jax.numpy as jnp
from jax import lax
from jax.experimental import pallas as pl
from jax.experimental.pallas import tpu as pltpu
from jax.experimental.pallas import tpu_sc as plsc

N_STEPS = 3
H = 64
CHUNK = 125


def _make_sc_scatter(n_nodes, n_edges):
    info = plsc.get_sparse_core_info()
    nc, ns = info.num_cores, info.num_subcores
    nw = nc * ns
    assert n_edges % (nw * CHUNK) == 0
    rows_per_w = n_edges // (nw * CHUNK)
    npt = 640
    npt_stride = 624
    assert (ns - 1) * npt_stride + npt == n_nodes

    mesh = plsc.VectorSubcoreMesh(core_axis_name="c", subcore_axis_name="s")

    nbuf = 4
    assert rows_per_w % nbuf == 0
    n_groups = rows_per_w // nbuf

    @functools.partial(
        pl.kernel,
        mesh=mesh,
        out_type=jax.ShapeDtypeStruct((nc * n_nodes, 2 * H), jnp.float32),
        scratch_types=[
            pltpu.VMEM((rows_per_w, CHUNK), jnp.int32),
            pltpu.VMEM((rows_per_w, CHUNK), jnp.int32),
            pltpu.VMEM((nbuf, CHUNK, H), jnp.float32),
            pltpu.VMEM((2, 64, H), jnp.float32),
            pltpu.VMEM_SHARED((n_nodes, H), jnp.float32),
            pltpu.SemaphoreType.DMA((nbuf,)),
            pltpu.SemaphoreType.DMA((nbuf,)),
            pltpu.SemaphoreType.DMA((2,)),
            pltpu.SemaphoreType.DMA,
        ],
        compiler_params=pltpu.CompilerParams(use_tc_tiling_on_sc=False),
    )
    def sc_scatter(table_hbm, src_hbm, dst_hbm, out_hbm,
                   src_v, dst_v, rows_v, wb_v, accum_sh, gsem, ssem, isem, zsem):
        c = lax.axis_index("c")
        s = lax.axis_index("s")
        wid = s * nc + c
        row0 = wid * rows_per_w
        i1 = pltpu.async_copy(src_hbm.at[pl.ds(row0, rows_per_w)], src_v,
                              isem.at[0])
        i2 = pltpu.async_copy(dst_hbm.at[pl.ds(row0, rows_per_w)], dst_v,
                              isem.at[1])

        zvec = jnp.zeros((16,), jnp.float32)

        def zrow(i, carry):
            for k in range(H // 16):
                wb_v[0, i, pl.ds(k * 16, 16)] = zvec
            return carry

        lax.fori_loop(0, 64, zrow, 0)
        zrow0 = s * npt_stride
        zs = [pltpu.async_copy(wb_v.at[0],
                               accum_sh.at[pl.ds(zrow0 + 64 * j, 64)], zsem)
              for j in range(npt // 64)]
        for z in zs:
            z.wait()
        i1.wait()
        i2.wait()
        plsc.subcore_barrier()

        def start_gather(j, b):
            pltpu.async_copy(table_hbm.at[src_v.at[j]], rows_v.at[b],
                             gsem.at[b])

        def wait_gather(j, b):
            pltpu.make_async_copy(table_hbm.at[src_v.at[j]], rows_v.at[b],
                                  gsem.at[b]).wait()

        def start_scatter(j, b):
            pltpu.async_copy(rows_v.at[b], accum_sh.at[dst_v.at[j]],
                             ssem.at[b], add=True)

        def wait_scatter(j, b):
            pltpu.make_async_copy(rows_v.at[b], accum_sh.at[dst_v.at[j]],
                                  ssem.at[b]).wait()

        def chunk_iter(j, b, first_group, last_group):
            b2 = (b + 2) % nbuf
            if not first_group or b >= 2:
                wait_scatter(j - 2, b2)
            if not last_group or b < 2:
                start_gather(j + 2, b2)
            wait_gather(j, b)
            start_scatter(j, b)

        start_gather(0, 0)
        start_gather(1, 1)
        for b in range(nbuf):
            chunk_iter(b, b, True, False)

        def group(g, carry):
            for b in range(nbuf):
                chunk_iter(g * nbuf + b, b, False, False)
            return carry

        lax.fori_loop(1, n_groups - 1, group, 0)
        for b in range(nbuf):
            chunk_iter((n_groups - 1) * nbuf + b, b, False, True)
        wait_scatter(rows_per_w - 2, 2)
        wait_scatter(rows_per_w - 1, 3)

        plsc.subcore_barrier()
        orow0 = c * n_nodes + zrow0
        wbs = []
        for j in range(npt // 64):
            b = j % 2
            if j >= 2:
                wbs[j - 2].wait()
            pltpu.sync_copy(accum_sh.at[pl.ds(zrow0 + 64 * j, 64)], wb_v.at[b])
            wbs.append(pltpu.async_copy(
                wb_v.at[b],
                out_hbm.at[pl.ds(orow0 + 64 * j, 64), pl.ds(0, H)],
                isem.at[b]))
        wbs[-2].wait()
        wbs[-1].wait()

    return sc_scatter


def _dg(a, w):
    return lax.dot_general(a, w, (((1,), (1,)), ((), ())),
                           preferred_element_type=jnp.float32)


def _enc_body(x_ref, we_ref, be_ref, wm_ref, bm_ref, h_ref, hw_ref):
    h = _dg(x_ref[...], we_ref[...]) + be_ref[...]
    h_ref[...] = h
    hw_ref[...] = _dg(h, wm_ref[...]) + bm_ref[...]


def _gru_body(p0_ref, p1_ref, h_ref, wir_ref, wiz_ref, win_ref,
              whr_ref, whz_ref, whn_ref, br_ref, bz_ref, bin_ref, bhn_ref,
              wm_ref, bm_ref, hn_ref, hw_ref):
    m = p0_ref[:, :H] + p1_ref[:, :H]
    h = h_ref[...]
    r = jax.nn.sigmoid(_dg(m, wir_ref[...]) + _dg(h, whr_ref[...]) + br_ref[...])
    z = jax.nn.sigmoid(_dg(m, wiz_ref[...]) + _dg(h, whz_ref[...]) + bz_ref[...])
    n = jnp.tanh(_dg(m, win_ref[...]) + bin_ref[...]
                 + r * (_dg(h, whn_ref[...]) + bhn_ref[...]))
    hn = (1.0 - z) * n + z * h
    hn_ref[...] = hn
    hw_ref[...] = _dg(hn, wm_ref[...]) + bm_ref[...]


def _readout_body(h_ref, w1_ref, b1_ref, w2_ref, b2_ref, o_ref):
    g = jnp.sum(h_ref[...], axis=0, keepdims=True)
    t = jnp.maximum(_dg(g, w1_ref[...]) + b1_ref[...], 0.0)
    o_ref[...] = _dg(t, w2_ref[...]) + b2_ref[...]


def kernel(x, edge_index, W_enc, b_enc, W_msg, b_msg, W_ih, b_ih, W_hh, b_hh,
           W_r1, b_r1, W_r2, b_r2):
    n_nodes, d_feat = x.shape
    n_edges = edge_index.shape[1]
    out_dim = W_r2.shape[0]

    info = plsc.get_sparse_core_info()
    nc = info.num_cores

    ei = edge_index.astype(jnp.int32)
    srcr = ei[0].reshape(-1, CHUNK)
    dstr = ei[1].reshape(-1, CHUNK)

    sc_scatter = _make_sc_scatter(n_nodes, n_edges)

    blk = 1000
    assert n_nodes % blk == 0
    grid = n_nodes // blk

    b_enc2 = b_enc.reshape(1, H)
    b_msg2 = b_msg.reshape(1, H)
    b_r = (b_ih[:H] + b_hh[:H]).reshape(1, H)
    b_z = (b_ih[H:2 * H] + b_hh[H:2 * H]).reshape(1, H)
    b_in = b_ih[2 * H:].reshape(1, H)
    b_hn = b_hh[2 * H:].reshape(1, H)
    W_ir, W_iz, W_in = W_ih[:H], W_ih[H:2 * H], W_ih[2 * H:]
    W_hr, W_hz, W_hn = W_hh[:H], W_hh[H:2 * H], W_hh[2 * H:]

    full = lambda shape: pl.BlockSpec(shape, lambda i: (0, 0))
    rows = lambda w: pl.BlockSpec((blk, w), lambda i: (i, 0))

    h, hw = pl.pallas_call(
        _enc_body,
        grid=(grid,),
        in_specs=[rows(d_feat), full((H, d_feat)), full((1, H)),
                  full((H, H)), full((1, H))],
        out_specs=[rows(H), rows(H)],
        out_shape=[jax.ShapeDtypeStruct((n_nodes, H), jnp.float32)] * 2,
    )(x, W_enc, b_enc2, W_msg, b_msg2)

    gru_call = pl.pallas_call(
        _gru_body,
        grid=(grid,),
        in_specs=[pl.BlockSpec((blk, 2 * H), lambda i: (i, 0)),
                  pl.BlockSpec((blk, 2 * H), lambda i: (i + grid, 0)),
                  rows(H)]
                 + [full((H, H))] * 6 + [full((1, H))] * 4
                 + [full((H, H)), full((1, H))],
        out_specs=[rows(H), rows(H)],
        out_shape=[jax.ShapeDtypeStruct((n_nodes, H), jnp.float32)] * 2,
    )

    for _ in range(N_STEPS):
        parts = sc_scatter(hw, srcr, dstr)
        h, hw = gru_call(parts, parts, h, W_ir, W_iz, W_in, W_hr, W_hz, W_hn,
                         b_r, b_z, b_in, b_hn, W_msg, b_msg2)

    out = pl.pallas_call(
        _readout_body,
        in_specs=[pl.BlockSpec((n_nodes, H), lambda: (0, 0)),
                  pl.BlockSpec((H, H), lambda: (0, 0)),
                  pl.BlockSpec((1, H), lambda: (0, 0)),
                  pl.BlockSpec((out_dim, H), lambda: (0, 0)),
                  pl.BlockSpec((1, out_dim), lambda: (0, 0))],
        out_specs=pl.BlockSpec((1, out_dim), lambda: (0, 0)),
        out_shape=jax.ShapeDtypeStruct((1, out_dim), jnp.float32),
    )(h, W_r1, b_r1.reshape(1, H), W_r2, b_r2.reshape(1, out_dim))
    return out


# speedup vs baseline: 16.0464x; 1.0499x over previous
import functools

import jax
import jax.numpy as jnp
from jax import lax
from jax.experimental import pallas as pl
from jax.experimental.pallas import tpu as pltpu
from jax.experimental.pallas import tpu_sc as plsc

N_STEPS = 3
H = 64
CHUNK = 125


def _make_sc_scatter(n_nodes, n_edges):
    info = plsc.get_sparse_core_info()
    nc, ns = info.num_cores, info.num_subcores
    nw = nc * ns
    assert n_edges % (nw * CHUNK) == 0
    rows_per_w = n_edges // (nw * CHUNK)
    npt = 640
    npt_stride = 624
    assert (ns - 1) * npt_stride + npt == n_nodes

    mesh = plsc.VectorSubcoreMesh(core_axis_name="c", subcore_axis_name="s")

    nbuf = 4
    assert rows_per_w % nbuf == 0
    n_groups = rows_per_w // nbuf

    @functools.partial(
        pl.kernel,
        mesh=mesh,
        out_type=jax.ShapeDtypeStruct((nc * n_nodes, 2 * H), jnp.float32),
        scratch_types=[
            pltpu.VMEM((rows_per_w, CHUNK), jnp.int32),
            pltpu.VMEM((rows_per_w, CHUNK), jnp.int32),
            pltpu.VMEM((nbuf, CHUNK, H), jnp.float32),
            pltpu.VMEM((2, 64, H), jnp.float32),
            pltpu.VMEM_SHARED((n_nodes, H), jnp.float32),
            pltpu.SemaphoreType.DMA((nbuf,)),
            pltpu.SemaphoreType.DMA((nbuf,)),
            pltpu.SemaphoreType.DMA((2,)),
            pltpu.SemaphoreType.DMA,
        ],
        compiler_params=pltpu.CompilerParams(use_tc_tiling_on_sc=False),
    )
    def sc_scatter(table_hbm, src_hbm, dst_hbm, out_hbm,
                   src_v, dst_v, rows_v, wb_v, accum_sh, gsem, ssem, isem, zsem):
        c = lax.axis_index("c")
        s = lax.axis_index("s")
        wid = s * nc + c
        row0 = wid * rows_per_w
        i1 = pltpu.async_copy(src_hbm.at[pl.ds(row0, rows_per_w)], src_v,
                              isem.at[0])
        i2 = pltpu.async_copy(dst_hbm.at[pl.ds(row0, rows_per_w)], dst_v,
                              isem.at[1])

        zvec = jnp.zeros((16,), jnp.float32)

        def zrow(i, carry):
            for k in range(H // 16):
                wb_v[0, i, pl.ds(k * 16, 16)] = zvec
            return carry

        lax.fori_loop(0, 64, zrow, 0)
        zrow0 = s * npt_stride
        zs = [pltpu.async_copy(wb_v.at[0],
                               accum_sh.at[pl.ds(zrow0 + 64 * j, 64)], zsem)
              for j in range(npt // 64)]
        for z in zs:
            z.wait()
        i1.wait()
        i2.wait()
        plsc.subcore_barrier()

        def start_gather(j, b):
            pltpu.async_copy(table_hbm.at[src_v.at[j]], rows_v.at[b],
                             gsem.at[b])

        def wait_gather(j, b):
            pltpu.make_async_copy(table_hbm.at[src_v.at[j]], rows_v.at[b],
                                  gsem.at[b]).wait()

        def start_scatter(j, b):
            pltpu.async_copy(rows_v.at[b], accum_sh.at[dst_v.at[j]],
                             ssem.at[b], add=True)

        def wait_scatter(j, b):
            pltpu.make_async_copy(rows_v.at[b], accum_sh.at[dst_v.at[j]],
                                  ssem.at[b]).wait()

        def chunk_iter(j, b, first_group, last_group):
            b2 = (b + 2) % nbuf
            if not first_group or b >= 2:
                wait_scatter(j - 2, b2)
            if not last_group or b < 2:
                start_gather(j + 2, b2)
            wait_gather(j, b)
            start_scatter(j, b)

        start_gather(0, 0)
        start_gather(1, 1)
        for b in range(nbuf):
            chunk_iter(b, b, True, False)

        def group(g, carry):
            for b in range(nbuf):
                chunk_iter(g * nbuf + b, b, False, False)
            return carry

        lax.fori_loop(1, n_groups - 1, group, 0)
        for b in range(nbuf):
            chunk_iter((n_groups - 1) * nbuf + b, b, False, True)
        wait_scatter(rows_per_w - 2, 2)
        wait_scatter(rows_per_w - 1, 3)

        plsc.subcore_barrier()
        orow0 = c * n_nodes + zrow0
        wbs = []
        for j in range(npt // 64):
            b = j % 2
            if j >= 2:
                wbs[j - 2].wait()
            pltpu.sync_copy(accum_sh.at[pl.ds(zrow0 + 64 * j, 64)], wb_v.at[b])
            wbs.append(pltpu.async_copy(
                wb_v.at[b],
                out_hbm.at[pl.ds(orow0 + 64 * j, 64), pl.ds(0, H)],
                isem.at[b]))
        wbs[-2].wait()
        wbs[-1].wait()

    return sc_scatter


def _dg(a, w):
    return lax.dot_general(a, w, (((1,), (1,)), ((), ())),
                           preferred_element_type=jnp.float32)


def _enc_body(x_ref, we_ref, be_ref, wm_ref, bm_ref, h_ref, hw_ref):
    h = _dg(x_ref[...], we_ref[...]) + be_ref[...]
    h_ref[...] = h
    hw = _dg(h, wm_ref[...]) + bm_ref[...]
    hw_ref[...] = jnp.concatenate([hw, hw], axis=1)


def _gru_body(p0_ref, p1_ref, h_ref, wir_ref, wiz_ref, win_ref,
              whr_ref, whz_ref, whn_ref, br_ref, bz_ref, bin_ref, bhn_ref,
              wm_ref, bm_ref, hn_ref, hw_ref):
    m = p0_ref[:, :H] + p1_ref[:, :H]
    h = h_ref[...]
    r = jax.nn.sigmoid(_dg(m, wir_ref[...]) + _dg(h, whr_ref[...]) + br_ref[...])
    z = jax.nn.sigmoid(_dg(m, wiz_ref[...]) + _dg(h, whz_ref[...]) + bz_ref[...])
    n = jnp.tanh(_dg(m, win_ref[...]) + bin_ref[...]
                 + r * (_dg(h, whn_ref[...]) + bhn_ref[...]))
    hn = (1.0 - z) * n + z * h
    hn_ref[...] = hn
    hw = _dg(hn, wm_ref[...]) + bm_ref[...]
    hw_ref[...] = jnp.concatenate([hw, hw], axis=1)


def _readout_body(h_ref, w1_ref, b1_ref, w2_ref, b2_ref, o_ref):
    g = jnp.sum(h_ref[...], axis=0, keepdims=True)
    t = jnp.maximum(_dg(g, w1_ref[...]) + b1_ref[...], 0.0)
    o_ref[...] = _dg(t, w2_ref[...]) + b2_ref[...]


def kernel(x, edge_index, W_enc, b_enc, W_msg, b_msg, W_ih, b_ih, W_hh, b_hh,
           W_r1, b_r1, W_r2, b_r2):
    n_nodes, d_feat = x.shape
    n_edges = edge_index.shape[1]
    out_dim = W_r2.shape[0]

    info = plsc.get_sparse_core_info()
    nc = info.num_cores

    ei = edge_index.astype(jnp.int32)
    srcr = (ei[0] * 2).reshape(-1, CHUNK)
    dstr = ei[1].reshape(-1, CHUNK)

    sc_scatter = _make_sc_scatter(n_nodes, n_edges)

    blk = 1000
    assert n_nodes % blk == 0
    grid = n_nodes // blk

    b_enc2 = b_enc.reshape(1, H)
    b_msg2 = b_msg.reshape(1, H)
    b_r = (b_ih[:H] + b_hh[:H]).reshape(1, H)
    b_z = (b_ih[H:2 * H] + b_hh[H:2 * H]).reshape(1, H)
    b_in = b_ih[2 * H:].reshape(1, H)
    b_hn = b_hh[2 * H:].reshape(1, H)
    W_ir, W_iz, W_in = W_ih[:H], W_ih[H:2 * H], W_ih[2 * H:]
    W_hr, W_hz, W_hn = W_hh[:H], W_hh[H:2 * H], W_hh[2 * H:]

    full = lambda shape: pl.BlockSpec(shape, lambda i: (0, 0))
    rows = lambda w: pl.BlockSpec((blk, w), lambda i: (i, 0))

    h, hw = pl.pallas_call(
        _enc_body,
        grid=(grid,),
        in_specs=[rows(d_feat), full((H, d_feat)), full((1, H)),
                  full((H, H)), full((1, H))],
        out_specs=[rows(H), rows(2 * H)],
        out_shape=[jax.ShapeDtypeStruct((n_nodes, H), jnp.float32),
                   jax.ShapeDtypeStruct((n_nodes, 2 * H), jnp.float32)],
    )(x, W_enc, b_enc2, W_msg, b_msg2)

    gru_call = pl.pallas_call(
        _gru_body,
        grid=(grid,),
        in_specs=[pl.BlockSpec((blk, 2 * H), lambda i: (i, 0)),
                  pl.BlockSpec((blk, 2 * H), lambda i: (i + grid, 0)),
                  rows(H)]
                 + [full((H, H))] * 6 + [full((1, H))] * 4
                 + [full((H, H)), full((1, H))],
        out_specs=[rows(H), rows(2 * H)],
        out_shape=[jax.ShapeDtypeStruct((n_nodes, H), jnp.float32),
                   jax.ShapeDtypeStruct((n_nodes, 2 * H), jnp.float32)],
    )

    for _ in range(N_STEPS):
        parts = sc_scatter(hw.reshape(2 * n_nodes, H), srcr, dstr)
        h, hw = gru_call(parts, parts, h, W_ir, W_iz, W_in, W_hr, W_hz, W_hn,
                         b_r, b_z, b_in, b_hn, W_msg, b_msg2)

    out = pl.pallas_call(
        _readout_body,
        in_specs=[pl.BlockSpec((n_nodes, H), lambda: (0, 0)),
                  pl.BlockSpec((H, H), lambda: (0, 0)),
                  pl.BlockSpec((1, H), lambda: (0, 0)),
                  pl.BlockSpec((out_dim, H), lambda: (0, 0)),
                  pl.BlockSpec((1, out_dim), lambda: (0, 0))],
        out_specs=pl.BlockSpec((1, out_dim), lambda: (0, 0)),
        out_shape=jax.ShapeDtypeStruct((1, out_dim), jnp.float32),
    )(h, W_r1, b_r1.reshape(1, H), W_r2, b_r2.reshape(1, out_dim))
    return out


# speedup vs baseline: 16.6900x; 1.0401x over previous
import functools

import jax
import jax.numpy as jnp
from jax import lax
from jax.experimental import pallas as pl
from jax.experimental.pallas import tpu as pltpu
from jax.experimental.pallas import tpu_sc as plsc

N_STEPS = 3
H = 64
CHUNK = 125


def _make_sc_scatter(n_nodes, n_edges):
    info = plsc.get_sparse_core_info()
    nc, ns = info.num_cores, info.num_subcores
    nw = nc * ns
    assert n_edges % (nw * CHUNK) == 0
    rows_per_w = n_edges // (nw * CHUNK)
    npt = 640
    npt_stride = 624
    assert (ns - 1) * npt_stride + npt == n_nodes

    mesh = plsc.VectorSubcoreMesh(core_axis_name="c", subcore_axis_name="s")

    nbuf = 4
    assert rows_per_w % nbuf == 0
    n_groups = rows_per_w // nbuf

    @functools.partial(
        pl.kernel,
        mesh=mesh,
        out_type=jax.ShapeDtypeStruct((nc * n_nodes, 2 * H), jnp.float32),
        scratch_types=[
            pltpu.VMEM((rows_per_w, CHUNK), jnp.int32),
            pltpu.VMEM((rows_per_w, CHUNK), jnp.int32),
            pltpu.VMEM((nbuf, CHUNK, H), jnp.float32),
            pltpu.VMEM((2, 64, H), jnp.float32),
            pltpu.VMEM_SHARED((n_nodes, H), jnp.float32),
            pltpu.SemaphoreType.DMA((nbuf,)),
            pltpu.SemaphoreType.DMA((nbuf,)),
            pltpu.SemaphoreType.DMA((2,)),
            pltpu.SemaphoreType.DMA,
        ],
        compiler_params=pltpu.CompilerParams(use_tc_tiling_on_sc=False),
    )
    def sc_scatter(table_hbm, src_hbm, dst_hbm, out_hbm,
                   src_v, dst_v, rows_v, wb_v, accum_sh, gsem, ssem, isem, zsem):
        c = lax.axis_index("c")
        s = lax.axis_index("s")
        wid = s * nc + c
        row0 = wid * rows_per_w
        i1 = pltpu.async_copy(src_hbm.at[pl.ds(row0, rows_per_w)], src_v,
                              isem.at[0])
        i2 = pltpu.async_copy(dst_hbm.at[pl.ds(row0, rows_per_w)], dst_v,
                              isem.at[1])

        zvec = jnp.zeros((16,), jnp.float32)

        def zrow(i, carry):
            for k in range(H // 16):
                wb_v[0, i, pl.ds(k * 16, 16)] = zvec
            return carry

        lax.fori_loop(0, 64, zrow, 0)
        zrow0 = s * npt_stride
        zs = [pltpu.async_copy(wb_v.at[0],
                               accum_sh.at[pl.ds(zrow0 + 64 * j, 64)], zsem)
              for j in range(npt // 64)]
        for z in zs:
            z.wait()
        i1.wait()
        i2.wait()
        plsc.subcore_barrier()

        def start_gather(j, b):
            pltpu.async_copy(table_hbm.at[src_v.at[j]], rows_v.at[b],
                             gsem.at[b])

        def wait_gather(j, b):
            pltpu.make_async_copy(table_hbm.at[src_v.at[j]], rows_v.at[b],
                                  gsem.at[b]).wait()

        def start_scatter(j, b):
            pltpu.async_copy(rows_v.at[b], accum_sh.at[dst_v.at[j]],
                             ssem.at[b], add=True)

        def wait_scatter(j, b):
            pltpu.make_async_copy(rows_v.at[b], accum_sh.at[dst_v.at[j]],
                                  ssem.at[b]).wait()

        def chunk_iter(j, b, first_group, last_group):
            b2 = (b + 2) % nbuf
            if not first_group or b >= 2:
                wait_scatter(j - 2, b2)
            if not last_group or b < 2:
                start_gather(j + 2, b2)
            wait_gather(j, b)
            start_scatter(j, b)

        start_gather(0, 0)
        start_gather(1, 1)
        for b in range(nbuf):
            chunk_iter(b, b, True, False)

        def group(g, carry):
            for b in range(nbuf):
                chunk_iter(g * nbuf + b, b, False, False)
            return carry

        lax.fori_loop(1, n_groups - 1, group, 0)
        for b in range(nbuf):
            chunk_iter((n_groups - 1) * nbuf + b, b, False, True)
        wait_scatter(rows_per_w - 2, 2)
        wait_scatter(rows_per_w - 1, 3)

        plsc.subcore_barrier()
        orow0 = c * n_nodes + zrow0
        wbs = []
        for j in range(npt // 64):
            b = j % 2
            if j >= 2:
                wbs[j - 2].wait()
            pltpu.sync_copy(accum_sh.at[pl.ds(zrow0 + 64 * j, 64)], wb_v.at[b])
            wbs.append(pltpu.async_copy(
                wb_v.at[b],
                out_hbm.at[pl.ds(orow0 + 64 * j, 64), pl.ds(0, H)],
                isem.at[b]))
        wbs[-2].wait()
        wbs[-1].wait()

    return sc_scatter


def _dg(a, w):
    return lax.dot_general(a, w, (((1,), (0,)), ((), ())),
                           preferred_element_type=jnp.float32)


def _enc_body(x_ref, we_ref, be_ref, wm2_ref, bm2_ref, h_ref, hw_ref):
    h = _dg(x_ref[...], we_ref[...]) + be_ref[...]
    h_ref[...] = h
    hw_ref[...] = _dg(h, wm2_ref[...]) + bm2_ref[...]


def _gates(p0_ref, p1_ref, h, wg_ref, bg_ref):
    m = p0_ref[:, :H] + p1_ref[:, :H]
    mh = jnp.concatenate([m, h], axis=1)
    g = _dg(mh, wg_ref[...]) + bg_ref[...]
    r = jax.nn.sigmoid(g[:, :H])
    z = jax.nn.sigmoid(g[:, H:2 * H])
    n = jnp.tanh(g[:, 2 * H:3 * H] + r * g[:, 3 * H:])
    return (1.0 - z) * n + z * h


def _gru_body(p0_ref, p1_ref, h_ref, wg_ref, bg_ref, wm2_ref, bm2_ref,
              hn_ref, hw_ref):
    hn = _gates(p0_ref, p1_ref, h_ref[...], wg_ref, bg_ref)
    hn_ref[...] = hn
    hw_ref[...] = _dg(hn, wm2_ref[...]) + bm2_ref[...]


def _gru_final_body(p0_ref, p1_ref, h_ref, wg_ref, bg_ref,
                    w1_ref, b1_ref, w2_ref, b2_ref, o_ref, gsum_ref):
    i = pl.program_id(0)
    hn = _gates(p0_ref, p1_ref, h_ref[...], wg_ref, bg_ref)
    part = jnp.sum(hn, axis=0, keepdims=True)

    @pl.when(i == 0)
    def _():
        gsum_ref[...] = part

    @pl.when(i > 0)
    def _():
        gsum_ref[...] += part

    @pl.when(i == pl.num_programs(0) - 1)
    def _():
        t = jnp.maximum(_dg(gsum_ref[...], w1_ref[...]) + b1_ref[...], 0.0)
        o_ref[...] = _dg(t, w2_ref[...]) + b2_ref[...]


def kernel(x, edge_index, W_enc, b_enc, W_msg, b_msg, W_ih, b_ih, W_hh, b_hh,
           W_r1, b_r1, W_r2, b_r2):
    n_nodes, d_feat = x.shape
    n_edges = edge_index.shape[1]
    out_dim = W_r2.shape[0]

    info = plsc.get_sparse_core_info()
    nc = info.num_cores

    ei = edge_index.astype(jnp.int32)
    srcr = (ei[0] * 2).reshape(-1, CHUNK)
    dstr = ei[1].reshape(-1, CHUNK)

    sc_scatter = _make_sc_scatter(n_nodes, n_edges)

    blk = 1000
    assert n_nodes % blk == 0
    grid = n_nodes // blk

    b_enc2 = b_enc.reshape(1, H)
    zz = jnp.zeros((H, H), jnp.float32)
    W_i = W_ih.T
    W_h = W_hh.T
    Wg = jnp.concatenate([
        jnp.concatenate([W_i[:, :2 * H], W_i[:, 2 * H:], zz], axis=1),
        jnp.concatenate([W_h[:, :2 * H], zz, W_h[:, 2 * H:]], axis=1),
    ], axis=0)
    bg = jnp.concatenate([b_ih[:2 * H] + b_hh[:2 * H], b_ih[2 * H:],
                          b_hh[2 * H:]]).reshape(1, 4 * H)
    Wm2 = jnp.concatenate([W_msg.T, W_msg.T], axis=1)
    bm2 = jnp.concatenate([b_msg, b_msg]).reshape(1, 2 * H)

    full = lambda shape: pl.BlockSpec(shape, lambda i: (0, 0))
    rows = lambda w: pl.BlockSpec((blk, w), lambda i: (i, 0))

    h, hw = pl.pallas_call(
        _enc_body,
        grid=(grid,),
        in_specs=[rows(d_feat), full((d_feat, H)), full((1, H)),
                  full((H, 2 * H)), full((1, 2 * H))],
        out_specs=[rows(H), rows(2 * H)],
        out_shape=[jax.ShapeDtypeStruct((n_nodes, H), jnp.float32),
                   jax.ShapeDtypeStruct((n_nodes, 2 * H), jnp.float32)],
    )(x, W_enc.T, b_enc2, Wm2, bm2)

    p_specs = [pl.BlockSpec((blk, 2 * H), lambda i: (i, 0)),
               pl.BlockSpec((blk, 2 * H), lambda i: (i + grid, 0))]
    gru_call = pl.pallas_call(
        _gru_body,
        grid=(grid,),
        in_specs=p_specs + [rows(H), full((2 * H, 4 * H)), full((1, 4 * H)),
                            full((H, 2 * H)), full((1, 2 * H))],
        out_specs=[rows(H), rows(2 * H)],
        out_shape=[jax.ShapeDtypeStruct((n_nodes, H), jnp.float32),
                   jax.ShapeDtypeStruct((n_nodes, 2 * H), jnp.float32)],
    )

    for _ in range(N_STEPS - 1):
        parts = sc_scatter(hw.reshape(2 * n_nodes, H), srcr, dstr)
        h, hw = gru_call(parts, parts, h, Wg, bg, Wm2, bm2)

    parts = sc_scatter(hw.reshape(2 * n_nodes, H), srcr, dstr)
    out = pl.pallas_call(
        _gru_final_body,
        grid=(grid,),
        in_specs=p_specs + [rows(H), full((2 * H, 4 * H)), full((1, 4 * H)),
                            full((H, H)), full((1, H)),
                            full((H, out_dim)), full((1, out_dim))],
        out_specs=pl.BlockSpec((1, out_dim), lambda i: (0, 0)),
        out_shape=jax.ShapeDtypeStruct((1, out_dim), jnp.float32),
        scratch_shapes=[pltpu.VMEM((1, H), jnp.float32)],
    )(parts, parts, h, Wg, bg, W_r1.T, b_r1.reshape(1, H),
      W_r2.T, b_r2.reshape(1, out_dim))
    return out


# speedup vs baseline: 16.7566x; 1.0040x over previous
import functools

import jax
import jax.numpy as jnp
from jax import lax
from jax.experimental import pallas as pl
from jax.experimental.pallas import tpu as pltpu
from jax.experimental.pallas import tpu_sc as plsc

N_STEPS = 3
H = 64
CHUNK = 128


def _make_sc_scatter(n_nodes, n_edges):
    info = plsc.get_sparse_core_info()
    nc, ns = info.num_cores, info.num_subcores
    nw = nc * ns
    assert n_edges % CHUNK == 0
    chunks_total = n_edges // CHUNK
    rows_per_w = chunks_total // nw
    leftover = chunks_total - nw * rows_per_w
    assert leftover < nw
    npt = 640
    npt_stride = 624
    assert (ns - 1) * npt_stride + npt == n_nodes

    mesh = plsc.VectorSubcoreMesh(core_axis_name="c", subcore_axis_name="s")

    nbuf = 4
    n_groups = rows_per_w // nbuf
    tail = rows_per_w - (n_groups - 1) * nbuf - nbuf

    @functools.partial(
        pl.kernel,
        mesh=mesh,
        out_type=jax.ShapeDtypeStruct((nc * n_nodes, 2 * H), jnp.float32),
        scratch_types=[
            pltpu.VMEM((rows_per_w, CHUNK), jnp.int32),
            pltpu.VMEM((rows_per_w, CHUNK), jnp.int32),
            pltpu.VMEM((2, CHUNK), jnp.int32),
            pltpu.VMEM((nbuf, CHUNK, H), jnp.float32),
            pltpu.VMEM((2, 64, H), jnp.float32),
            pltpu.VMEM_SHARED((n_nodes, H), jnp.float32),
            pltpu.SemaphoreType.DMA((nbuf,)),
            pltpu.SemaphoreType.DMA((nbuf,)),
            pltpu.SemaphoreType.DMA((2,)),
            pltpu.SemaphoreType.DMA,
        ],
        compiler_params=pltpu.CompilerParams(use_tc_tiling_on_sc=False),
    )
    def sc_scatter(table_hbm, src_hbm, dst_hbm, out_hbm,
                   src_v, dst_v, x_v, rows_v, wb_v, accum_sh,
                   gsem, ssem, isem, zsem):
        c = lax.axis_index("c")
        s = lax.axis_index("s")
        wid = s * nc + c
        row0 = wid * rows_per_w
        i1 = pltpu.async_copy(src_hbm.at[pl.ds(row0, rows_per_w)], src_v,
                              isem.at[0])
        i2 = pltpu.async_copy(dst_hbm.at[pl.ds(row0, rows_per_w)], dst_v,
                              isem.at[1])

        zvec = jnp.zeros((16,), jnp.float32)

        def zrow(i, carry):
            for k in range(H // 16):
                wb_v[0, i, pl.ds(k * 16, 16)] = zvec
            return carry

        lax.fori_loop(0, 64, zrow, 0)
        zrow0 = s * npt_stride
        zs = [pltpu.async_copy(wb_v.at[0],
                               accum_sh.at[pl.ds(zrow0 + 64 * j, 64)], zsem)
              for j in range(npt // 64)]
        for z in zs:
            z.wait()
        i1.wait()
        i2.wait()
        plsc.subcore_barrier()

        def start_gather(j, b):
            pltpu.async_copy(table_hbm.at[src_v.at[j]], rows_v.at[b],
                             gsem.at[b])

        def wait_gather(j, b):
            pltpu.make_async_copy(table_hbm.at[src_v.at[j]], rows_v.at[b],
                                  gsem.at[b]).wait()

        def start_scatter(j, b):
            pltpu.async_copy(rows_v.at[b], accum_sh.at[dst_v.at[j]],
                             ssem.at[b], add=True)

        def wait_scatter(j, b):
            pltpu.make_async_copy(rows_v.at[b], accum_sh.at[dst_v.at[j]],
                                  ssem.at[b]).wait()

        def chunk_iter(j, b, do_wait, do_start):
            b2 = (b + 2) % nbuf
            if do_wait:
                wait_scatter(j - 2, b2)
            if do_start:
                start_gather(j + 2, b2)
            wait_gather(j, b)
            start_scatter(j, b)

        start_gather(0, 0)
        start_gather(1, 1)
        for b in range(nbuf):
            chunk_iter(b, b, b >= 2, True)

        def group(g, carry):
            for b in range(nbuf):
                chunk_iter(g * nbuf + b, b, True, True)
            return carry

        lax.fori_loop(1, n_groups - 1, group, 0)
        for t in range(nbuf + tail):
            j = (n_groups - 1) * nbuf + t
            chunk_iter(j, j % nbuf, True, j + 2 < rows_per_w)
        wait_scatter(rows_per_w - 2, (rows_per_w - 2) % nbuf)
        wait_scatter(rows_per_w - 1, (rows_per_w - 1) % nbuf)

        @pl.when(wid < leftover)
        def _():
            xrow = nw * rows_per_w + wid
            pltpu.sync_copy(src_hbm.at[pl.ds(xrow, 1)], x_v.at[pl.ds(0, 1)])
            pltpu.sync_copy(dst_hbm.at[pl.ds(xrow, 1)], x_v.at[pl.ds(1, 1)])
            pltpu.async_copy(table_hbm.at[x_v.at[0]], rows_v.at[0],
                             gsem.at[0]).wait()
            pltpu.sync_copy(rows_v.at[0], accum_sh.at[x_v.at[1]], add=True)

        plsc.subcore_barrier()
        orow0 = c * n_nodes + zrow0
        wbs = []
        for j in range(npt // 64):
            b = j % 2
            if j >= 2:
                wbs[j - 2].wait()
            pltpu.sync_copy(accum_sh.at[pl.ds(zrow0 + 64 * j, 64)], wb_v.at[b])
            wbs.append(pltpu.async_copy(
                wb_v.at[b],
                out_hbm.at[pl.ds(orow0 + 64 * j, 64), pl.ds(0, H)],
                isem.at[b]))
        wbs[-2].wait()
        wbs[-1].wait()

    return sc_scatter


def _dg(a, w):
    return lax.dot_general(a, w, (((1,), (0,)), ((), ())),
                           preferred_element_type=jnp.float32)


def _enc_body(x_ref, we_ref, be_ref, wm2_ref, bm2_ref, h_ref, hw_ref):
    h = _dg(x_ref[...], we_ref[...]) + be_ref[...]
    h_ref[...] = h
    hw_ref[...] = _dg(h, wm2_ref[...]) + bm2_ref[...]


def _gates(p0_ref, p1_ref, h, wg_ref, bg_ref):
    m = p0_ref[:, :H] + p1_ref[:, :H]
    mh = jnp.concatenate([m, h], axis=1)
    g = _dg(mh, wg_ref[...]) + bg_ref[...]
    r = jax.nn.sigmoid(g[:, :H])
    z = jax.nn.sigmoid(g[:, H:2 * H])
    n = jnp.tanh(g[:, 2 * H:3 * H] + r * g[:, 3 * H:])
    return (1.0 - z) * n + z * h


def _gru_body(p0_ref, p1_ref, h_ref, wg_ref, bg_ref, wm2_ref, bm2_ref,
              hn_ref, hw_ref):
    hn = _gates(p0_ref, p1_ref, h_ref[...], wg_ref, bg_ref)
    hn_ref[...] = hn
    hw_ref[...] = _dg(hn, wm2_ref[...]) + bm2_ref[...]


def _gru_final_body(p0_ref, p1_ref, h_ref, wg_ref, bg_ref,
                    w1_ref, b1_ref, w2_ref, b2_ref, o_ref, gsum_ref):
    i = pl.program_id(0)
    hn = _gates(p0_ref, p1_ref, h_ref[...], wg_ref, bg_ref)
    part = jnp.sum(hn, axis=0, keepdims=True)

    @pl.when(i == 0)
    def _():
        gsum_ref[...] = part

    @pl.when(i > 0)
    def _():
        gsum_ref[...] += part

    @pl.when(i == pl.num_programs(0) - 1)
    def _():
        t = jnp.maximum(_dg(gsum_ref[...], w1_ref[...]) + b1_ref[...], 0.0)
        o_ref[...] = _dg(t, w2_ref[...]) + b2_ref[...]


def kernel(x, edge_index, W_enc, b_enc, W_msg, b_msg, W_ih, b_ih, W_hh, b_hh,
           W_r1, b_r1, W_r2, b_r2):
    n_nodes, d_feat = x.shape
    n_edges = edge_index.shape[1]
    out_dim = W_r2.shape[0]

    info = plsc.get_sparse_core_info()
    nc = info.num_cores

    ei = edge_index.astype(jnp.int32)
    srcr = (ei[0] * 2).reshape(-1, CHUNK)
    dstr = ei[1].reshape(-1, CHUNK)

    sc_scatter = _make_sc_scatter(n_nodes, n_edges)

    blk = 1000
    assert n_nodes % blk == 0
    grid = n_nodes // blk

    b_enc2 = b_enc.reshape(1, H)
    zz = jnp.zeros((H, H), jnp.float32)
    W_i = W_ih.T
    W_h = W_hh.T
    Wg = jnp.concatenate([
        jnp.concatenate([W_i[:, :2 * H], W_i[:, 2 * H:], zz], axis=1),
        jnp.concatenate([W_h[:, :2 * H], zz, W_h[:, 2 * H:]], axis=1),
    ], axis=0)
    bg = jnp.concatenate([b_ih[:2 * H] + b_hh[:2 * H], b_ih[2 * H:],
                          b_hh[2 * H:]]).reshape(1, 4 * H)
    Wm2 = jnp.concatenate([W_msg.T, W_msg.T], axis=1)
    bm2 = jnp.concatenate([b_msg, b_msg]).reshape(1, 2 * H)

    full = lambda shape: pl.BlockSpec(shape, lambda i: (0, 0))
    rows = lambda w: pl.BlockSpec((blk, w), lambda i: (i, 0))

    h, hw = pl.pallas_call(
        _enc_body,
        grid=(grid,),
        in_specs=[rows(d_feat), full((d_feat, H)), full((1, H)),
                  full((H, 2 * H)), full((1, 2 * H))],
        out_specs=[rows(H), rows(2 * H)],
        out_shape=[jax.ShapeDtypeStruct((n_nodes, H), jnp.float32),
                   jax.ShapeDtypeStruct((n_nodes, 2 * H), jnp.float32)],
    )(x, W_enc.T, b_enc2, Wm2, bm2)

    p_specs = [pl.BlockSpec((blk, 2 * H), lambda i: (i, 0)),
               pl.BlockSpec((blk, 2 * H), lambda i: (i + grid, 0))]
    gru_call = pl.pallas_call(
        _gru_body,
        grid=(grid,),
        in_specs=p_specs + [rows(H), full((2 * H, 4 * H)), full((1, 4 * H)),
                            full((H, 2 * H)), full((1, 2 * H))],
        out_specs=[rows(H), rows(2 * H)],
        out_shape=[jax.ShapeDtypeStruct((n_nodes, H), jnp.float32),
                   jax.ShapeDtypeStruct((n_nodes, 2 * H), jnp.float32)],
    )

    for _ in range(N_STEPS - 1):
        parts = sc_scatter(hw.reshape(2 * n_nodes, H), srcr, dstr)
        h, hw = gru_call(parts, parts, h, Wg, bg, Wm2, bm2)

    parts = sc_scatter(hw.reshape(2 * n_nodes, H), srcr, dstr)
    out = pl.pallas_call(
        _gru_final_body,
        grid=(grid,),
        in_specs=p_specs + [rows(H), full((2 * H, 4 * H)), full((1, 4 * H)),
                            full((H, H)), full((1, H)),
                            full((H, out_dim)), full((1, out_dim))],
        out_specs=pl.BlockSpec((1, out_dim), lambda i: (0, 0)),
        out_shape=jax.ShapeDtypeStruct((1, out_dim), jnp.float32),
        scratch_shapes=[pltpu.VMEM((1, H), jnp.float32)],
    )(parts, parts, h, Wg, bg, W_r1.T, b_r1.reshape(1, H),
      W_r2.T, b_r2.reshape(1, out_dim))
    return out


# speedup vs baseline: 16.9570x; 1.0120x over previous
import functools

import jax
import jax.numpy as jnp
from jax import lax
from jax.experimental import pallas as pl
from jax.experimental.pallas import tpu as pltpu
from jax.experimental.pallas import tpu_sc as plsc

N_STEPS = 3
H = 64
CHUNK = 128


def _make_sc_scatter(n_nodes, n_edges):
    info = plsc.get_sparse_core_info()
    nc, ns = info.num_cores, info.num_subcores
    nw = nc * ns
    assert n_edges % CHUNK == 0
    chunks_total = n_edges // CHUNK
    rows_per_w = chunks_total // nw
    leftover = chunks_total - nw * rows_per_w
    assert leftover < nw
    npt = 640
    npt_stride = 624
    assert (ns - 1) * npt_stride + npt == n_nodes

    mesh = plsc.VectorSubcoreMesh(core_axis_name="c", subcore_axis_name="s")

    nbuf = 4
    n_groups = rows_per_w // nbuf
    tail = rows_per_w - (n_groups - 1) * nbuf - nbuf

    @functools.partial(
        pl.kernel,
        mesh=mesh,
        out_type=jax.ShapeDtypeStruct((nc * n_nodes, 2 * H), jnp.float32),
        scratch_types=[
            pltpu.VMEM((rows_per_w, CHUNK), jnp.int32),
            pltpu.VMEM((rows_per_w, CHUNK), jnp.int32),
            pltpu.VMEM((2, CHUNK), jnp.int32),
            pltpu.VMEM((nbuf, CHUNK, H), jnp.float32),
            pltpu.VMEM((2, 64, H), jnp.float32),
            pltpu.VMEM_SHARED((n_nodes, H), jnp.float32),
            pltpu.SemaphoreType.DMA((nbuf,)),
            pltpu.SemaphoreType.DMA((nbuf,)),
            pltpu.SemaphoreType.DMA((2,)),
            pltpu.SemaphoreType.DMA,
        ],
        compiler_params=pltpu.CompilerParams(use_tc_tiling_on_sc=False),
    )
    def sc_scatter(table_hbm, src_hbm, dst_hbm, out_hbm,
                   src_v, dst_v, x_v, rows_v, wb_v, accum_sh,
                   gsem, ssem, isem, zsem):
        c = lax.axis_index("c")
        s = lax.axis_index("s")
        wid = s * nc + c
        row0 = wid * rows_per_w
        i1 = pltpu.async_copy(src_hbm.at[pl.ds(row0, rows_per_w)], src_v,
                              isem.at[0])
        i2 = pltpu.async_copy(dst_hbm.at[pl.ds(row0, rows_per_w)], dst_v,
                              isem.at[1])

        zvec = jnp.zeros((16,), jnp.float32)

        def zrow(i, carry):
            for k in range(H // 16):
                wb_v[0, i, pl.ds(k * 16, 16)] = zvec
            return carry

        lax.fori_loop(0, 64, zrow, 0)
        zrow0 = s * npt_stride
        zs = [pltpu.async_copy(wb_v.at[0],
                               accum_sh.at[pl.ds(zrow0 + 64 * j, 64)], zsem)
              for j in range(npt // 64)]
        for z in zs:
            z.wait()
        i1.wait()
        i2.wait()
        plsc.subcore_barrier()

        def start_gather(j, b):
            pltpu.async_copy(table_hbm.at[src_v.at[j]], rows_v.at[b],
                             gsem.at[b])

        def wait_gather(j, b):
            pltpu.make_async_copy(table_hbm.at[src_v.at[j]], rows_v.at[b],
                                  gsem.at[b]).wait()

        def start_scatter(j, b):
            pltpu.async_copy(rows_v.at[b], accum_sh.at[dst_v.at[j]],
                             ssem.at[b], add=True)

        def wait_scatter(j, b):
            pltpu.make_async_copy(rows_v.at[b], accum_sh.at[dst_v.at[j]],
                                  ssem.at[b]).wait()

        def chunk_iter(j, b, do_wait, do_start):
            b2 = (b + 2) % nbuf
            if do_wait:
                wait_scatter(j - 2, b2)
            if do_start:
                start_gather(j + 2, b2)
            wait_gather(j, b)
            start_scatter(j, b)

        start_gather(0, 0)
        start_gather(1, 1)
        for b in range(nbuf):
            chunk_iter(b, b, b >= 2, True)

        def group(g, carry):
            for b in range(nbuf):
                chunk_iter(g * nbuf + b, b, True, True)
            return carry

        lax.fori_loop(1, n_groups - 1, group, 0)
        for t in range(nbuf + tail):
            j = (n_groups - 1) * nbuf + t
            chunk_iter(j, j % nbuf, True, j + 2 < rows_per_w)
        wait_scatter(rows_per_w - 2, (rows_per_w - 2) % nbuf)
        wait_scatter(rows_per_w - 1, (rows_per_w - 1) % nbuf)

        @pl.when(wid < leftover)
        def _():
            xrow = nw * rows_per_w + wid
            pltpu.sync_copy(src_hbm.at[pl.ds(xrow, 1)], x_v.at[pl.ds(0, 1)])
            pltpu.sync_copy(dst_hbm.at[pl.ds(xrow, 1)], x_v.at[pl.ds(1, 1)])
            pltpu.async_copy(table_hbm.at[x_v.at[0]], rows_v.at[0],
                             gsem.at[0]).wait()
            pltpu.sync_copy(rows_v.at[0], accum_sh.at[x_v.at[1]], add=True)

        plsc.subcore_barrier()
        orow0 = c * n_nodes + zrow0
        wbs = []
        for j in range(npt // 64):
            b = j % 2
            if j >= 2:
                wbs[j - 2].wait()
            pltpu.sync_copy(accum_sh.at[pl.ds(zrow0 + 64 * j, 64)], wb_v.at[b])
            wbs.append(pltpu.async_copy(
                wb_v.at[b],
                out_hbm.at[pl.ds(orow0 + 64 * j, 64), pl.ds(0, H)],
                isem.at[b]))
        wbs[-2].wait()
        wbs[-1].wait()

    return sc_scatter


def _dg(a, w):
    return lax.dot_general(a, w, (((1,), (1,)), ((), ())),
                           preferred_element_type=jnp.float32)


def _enc_body(x_ref, we_ref, be_ref, wm2_ref, bm2_ref, h_ref, hw_ref):
    h = _dg(x_ref[...], we_ref[...]) + be_ref[...]
    h_ref[...] = h
    hw_ref[...] = _dg(h, wm2_ref[...]) + bm2_ref[...]


def _gates(p0_ref, p1_ref, h, wg_ref, bg_ref):
    m = p0_ref[:, :H] + p1_ref[:, :H]
    mh = jnp.concatenate([m, h], axis=1)
    g = _dg(mh, wg_ref[...]) + bg_ref[...]
    r = jax.nn.sigmoid(g[:, :H])
    z = jax.nn.sigmoid(g[:, H:2 * H])
    n = jnp.tanh(g[:, 2 * H:3 * H] + r * g[:, 3 * H:])
    return (1.0 - z) * n + z * h


def _gru_body(p0_ref, p1_ref, h_ref, wg_ref, bg_ref, wm2_ref, bm2_ref,
              hn_ref, hw_ref):
    hn = _gates(p0_ref, p1_ref, h_ref[...], wg_ref, bg_ref)
    hn_ref[...] = hn
    hw_ref[...] = _dg(hn, wm2_ref[...]) + bm2_ref[...]


def _gru_final_body(p0_ref, p1_ref, h_ref, wg_ref, bg_ref,
                    w1_ref, b1_ref, w2_ref, b2_ref, o_ref, gsum_ref):
    i = pl.program_id(0)
    hn = _gates(p0_ref, p1_ref, h_ref[...], wg_ref, bg_ref)
    part = jnp.sum(hn, axis=0, keepdims=True)

    @pl.when(i == 0)
    def _():
        gsum_ref[...] = part

    @pl.when(i > 0)
    def _():
        gsum_ref[...] += part

    @pl.when(i == pl.num_programs(0) - 1)
    def _():
        t = jnp.maximum(_dg(gsum_ref[...], w1_ref[...]) + b1_ref[...], 0.0)
        o_ref[...] = _dg(t, w2_ref[...]) + b2_ref[...]


def kernel(x, edge_index, W_enc, b_enc, W_msg, b_msg, W_ih, b_ih, W_hh, b_hh,
           W_r1, b_r1, W_r2, b_r2):
    n_nodes, d_feat = x.shape
    n_edges = edge_index.shape[1]
    out_dim = W_r2.shape[0]

    info = plsc.get_sparse_core_info()
    nc = info.num_cores

    ei = edge_index.astype(jnp.int32)
    srcr = (ei[0] * 2).reshape(-1, CHUNK)
    dstr = ei[1].reshape(-1, CHUNK)

    sc_scatter = _make_sc_scatter(n_nodes, n_edges)

    blk = 1000
    assert n_nodes % blk == 0
    grid = n_nodes // blk

    b_enc2 = b_enc.reshape(1, H)
    zz = jnp.zeros((H, H), jnp.float32)
    Wg = jnp.concatenate([
        jnp.concatenate([W_ih[:2 * H], W_hh[:2 * H]], axis=1),
        jnp.concatenate([W_ih[2 * H:], zz], axis=1),
        jnp.concatenate([zz, W_hh[2 * H:]], axis=1),
    ], axis=0)
    bg = jnp.concatenate([b_ih[:2 * H] + b_hh[:2 * H], b_ih[2 * H:],
                          b_hh[2 * H:]]).reshape(1, 4 * H)
    Wm2 = jnp.concatenate([W_msg, W_msg], axis=0)
    bm2 = jnp.concatenate([b_msg, b_msg]).reshape(1, 2 * H)

    full = lambda shape: pl.BlockSpec(shape, lambda i: (0, 0))
    rows = lambda w: pl.BlockSpec((blk, w), lambda i: (i, 0))

    h, hw = pl.pallas_call(
        _enc_body,
        grid=(grid,),
        in_specs=[rows(d_feat), full((H, d_feat)), full((1, H)),
                  full((2 * H, H)), full((1, 2 * H))],
        out_specs=[rows(H), rows(2 * H)],
        out_shape=[jax.ShapeDtypeStruct((n_nodes, H), jnp.float32),
                   jax.ShapeDtypeStruct((n_nodes, 2 * H), jnp.float32)],
    )(x, W_enc, b_enc2, Wm2, bm2)

    p_specs = [pl.BlockSpec((blk, 2 * H), lambda i: (i, 0)),
               pl.BlockSpec((blk, 2 * H), lambda i: (i + grid, 0))]
    gru_call = pl.pallas_call(
        _gru_body,
        grid=(grid,),
        in_specs=p_specs + [rows(H), full((4 * H, 2 * H)), full((1, 4 * H)),
                            full((2 * H, H)), full((1, 2 * H))],
        out_specs=[rows(H), rows(2 * H)],
        out_shape=[jax.ShapeDtypeStruct((n_nodes, H), jnp.float32),
                   jax.ShapeDtypeStruct((n_nodes, 2 * H), jnp.float32)],
    )

    for _ in range(N_STEPS - 1):
        parts = sc_scatter(hw.reshape(2 * n_nodes, H), srcr, dstr)
        h, hw = gru_call(parts, parts, h, Wg, bg, Wm2, bm2)

    parts = sc_scatter(hw.reshape(2 * n_nodes, H), srcr, dstr)
    out = pl.pallas_call(
        _gru_final_body,
        grid=(grid,),
        in_specs=p_specs + [rows(H), full((4 * H, 2 * H)), full((1, 4 * H)),
                            full((H, H)), full((1, H)),
                            full((out_dim, H)), full((1, out_dim))],
        out_specs=pl.BlockSpec((1, out_dim), lambda i: (0, 0)),
        out_shape=jax.ShapeDtypeStruct((1, out_dim), jnp.float32),
        scratch_shapes=[pltpu.VMEM((1, H), jnp.float32)],
    )(parts, parts, h, Wg, bg, W_r1, b_r1.reshape(1, H),
      W_r2, b_r2.reshape(1, out_dim))
    return out


# speedup vs baseline: 17.1486x; 1.0113x over previous
import functools

import jax
import jax.numpy as jnp
from jax import lax
from jax.experimental import pallas as pl
from jax.experimental.pallas import tpu as pltpu
from jax.experimental.pallas import tpu_sc as plsc

N_STEPS = 3
H = 64
CHUNK = 128


def _make_sc_scatter(n_nodes, n_edges):
    info = plsc.get_sparse_core_info()
    nc, ns = info.num_cores, info.num_subcores
    nw = nc * ns
    assert n_edges % CHUNK == 0
    chunks_total = n_edges // CHUNK
    rows_per_w = chunks_total // nw
    leftover = chunks_total - nw * rows_per_w
    assert leftover < nw
    npt = 640
    npt_stride = 624
    assert (ns - 1) * npt_stride + npt == n_nodes

    mesh = plsc.VectorSubcoreMesh(core_axis_name="c", subcore_axis_name="s")

    nbuf = 6
    n_groups = rows_per_w // nbuf
    tail = rows_per_w - (n_groups - 1) * nbuf - nbuf

    @functools.partial(
        pl.kernel,
        mesh=mesh,
        out_type=jax.ShapeDtypeStruct((nc * n_nodes, 2 * H), jnp.float32),
        scratch_types=[
            pltpu.VMEM((rows_per_w, CHUNK), jnp.int32),
            pltpu.VMEM((rows_per_w, CHUNK), jnp.int32),
            pltpu.VMEM((2, CHUNK), jnp.int32),
            pltpu.VMEM((nbuf, CHUNK, H), jnp.float32),
            pltpu.VMEM((2, 64, H), jnp.float32),
            pltpu.VMEM_SHARED((n_nodes, H), jnp.float32),
            pltpu.SemaphoreType.DMA((nbuf,)),
            pltpu.SemaphoreType.DMA((nbuf,)),
            pltpu.SemaphoreType.DMA((2,)),
            pltpu.SemaphoreType.DMA,
        ],
        compiler_params=pltpu.CompilerParams(use_tc_tiling_on_sc=False),
    )
    def sc_scatter(table_hbm, src_hbm, dst_hbm, out_hbm,
                   src_v, dst_v, x_v, rows_v, wb_v, accum_sh,
                   gsem, ssem, isem, zsem):
        c = lax.axis_index("c")
        s = lax.axis_index("s")
        wid = s * nc + c
        row0 = wid * rows_per_w
        i1 = pltpu.async_copy(src_hbm.at[pl.ds(row0, rows_per_w)], src_v,
                              isem.at[0])
        i2 = pltpu.async_copy(dst_hbm.at[pl.ds(row0, rows_per_w)], dst_v,
                              isem.at[1])

        zvec = jnp.zeros((16,), jnp.float32)

        def zrow(i, carry):
            for k in range(H // 16):
                wb_v[0, i, pl.ds(k * 16, 16)] = zvec
            return carry

        lax.fori_loop(0, 64, zrow, 0)
        zrow0 = s * npt_stride
        zs = [pltpu.async_copy(wb_v.at[0],
                               accum_sh.at[pl.ds(zrow0 + 64 * j, 64)], zsem)
              for j in range(npt // 64)]
        for z in zs:
            z.wait()
        i1.wait()
        i2.wait()
        plsc.subcore_barrier()

        def start_gather(j, b):
            pltpu.async_copy(table_hbm.at[src_v.at[j]], rows_v.at[b],
                             gsem.at[b])

        def wait_gather(j, b):
            pltpu.make_async_copy(table_hbm.at[src_v.at[j]], rows_v.at[b],
                                  gsem.at[b]).wait()

        def start_scatter(j, b):
            pltpu.async_copy(rows_v.at[b], accum_sh.at[dst_v.at[j]],
                             ssem.at[b], add=True)

        def wait_scatter(j, b):
            pltpu.make_async_copy(rows_v.at[b], accum_sh.at[dst_v.at[j]],
                                  ssem.at[b]).wait()

        def chunk_iter(j, b, do_wait, do_start):
            b2 = (b + 2) % nbuf
            if do_wait:
                wait_scatter(j + 2 - nbuf, b2)
            if do_start:
                start_gather(j + 2, b2)
            wait_gather(j, b)
            start_scatter(j, b)

        start_gather(0, 0)
        start_gather(1, 1)
        for b in range(nbuf):
            chunk_iter(b, b, b + 2 >= nbuf, True)

        def group(g, carry):
            for b in range(nbuf):
                chunk_iter(g * nbuf + b, b, True, True)
            return carry

        lax.fori_loop(1, n_groups - 1, group, 0)
        for t in range(nbuf + tail):
            j = (n_groups - 1) * nbuf + t
            chunk_iter(j, j % nbuf, True, j + 2 < rows_per_w)
        for j in range(rows_per_w + 2 - nbuf, rows_per_w):
            wait_scatter(j, j % nbuf)

        @pl.when(wid < leftover)
        def _():
            xrow = nw * rows_per_w + wid
            pltpu.sync_copy(src_hbm.at[pl.ds(xrow, 1)], x_v.at[pl.ds(0, 1)])
            pltpu.sync_copy(dst_hbm.at[pl.ds(xrow, 1)], x_v.at[pl.ds(1, 1)])
            pltpu.async_copy(table_hbm.at[x_v.at[0]], rows_v.at[0],
                             gsem.at[0]).wait()
            pltpu.sync_copy(rows_v.at[0], accum_sh.at[x_v.at[1]], add=True)

        plsc.subcore_barrier()
        orow0 = c * n_nodes + zrow0
        wbs = []
        for j in range(npt // 64):
            b = j % 2
            if j >= 2:
                wbs[j - 2].wait()
            pltpu.sync_copy(accum_sh.at[pl.ds(zrow0 + 64 * j, 64)], wb_v.at[b])
            wbs.append(pltpu.async_copy(
                wb_v.at[b],
                out_hbm.at[pl.ds(orow0 + 64 * j, 64), pl.ds(0, H)],
                isem.at[b]))
        wbs[-2].wait()
        wbs[-1].wait()

    return sc_scatter


def _dg(a, w):
    return lax.dot_general(a, w, (((1,), (1,)), ((), ())),
                           preferred_element_type=jnp.float32)


def _enc_body(x_ref, we_ref, be_ref, wm2_ref, bm2_ref, h_ref, hw_ref):
    h = _dg(x_ref[...], we_ref[...]) + be_ref[...]
    h_ref[...] = h
    hw_ref[...] = _dg(h, wm2_ref[...]) + bm2_ref[...]


def _gates(p0_ref, p1_ref, h, wg_ref, bg_ref):
    m = p0_ref[:, :H] + p1_ref[:, :H]
    mh = jnp.concatenate([m, h], axis=1)
    g = _dg(mh, wg_ref[...]) + bg_ref[...]
    r = jax.nn.sigmoid(g[:, :H])
    z = jax.nn.sigmoid(g[:, H:2 * H])
    n = jnp.tanh(g[:, 2 * H:3 * H] + r * g[:, 3 * H:])
    return (1.0 - z) * n + z * h


def _gru_body(p0_ref, p1_ref, h_ref, wg_ref, bg_ref, wm2_ref, bm2_ref,
              hn_ref, hw_ref):
    hn = _gates(p0_ref, p1_ref, h_ref[...], wg_ref, bg_ref)
    hn_ref[...] = hn
    hw_ref[...] = _dg(hn, wm2_ref[...]) + bm2_ref[...]


def _gru_final_body(p0_ref, p1_ref, h_ref, wg_ref, bg_ref,
                    w1_ref, b1_ref, w2_ref, b2_ref, o_ref, gsum_ref):
    i = pl.program_id(0)
    hn = _gates(p0_ref, p1_ref, h_ref[...], wg_ref, bg_ref)
    part = jnp.sum(hn, axis=0, keepdims=True)

    @pl.when(i == 0)
    def _():
        gsum_ref[...] = part

    @pl.when(i > 0)
    def _():
        gsum_ref[...] += part

    @pl.when(i == pl.num_programs(0) - 1)
    def _():
        t = jnp.maximum(_dg(gsum_ref[...], w1_ref[...]) + b1_ref[...], 0.0)
        o_ref[...] = _dg(t, w2_ref[...]) + b2_ref[...]


def kernel(x, edge_index, W_enc, b_enc, W_msg, b_msg, W_ih, b_ih, W_hh, b_hh,
           W_r1, b_r1, W_r2, b_r2):
    n_nodes, d_feat = x.shape
    n_edges = edge_index.shape[1]
    out_dim = W_r2.shape[0]

    info = plsc.get_sparse_core_info()
    nc = info.num_cores

    ei = edge_index.astype(jnp.int32)
    srcr = (ei[0] * 2).reshape(-1, CHUNK)
    dstr = ei[1].reshape(-1, CHUNK)

    sc_scatter = _make_sc_scatter(n_nodes, n_edges)

    blk = 2000
    assert n_nodes % blk == 0
    grid = n_nodes // blk

    b_enc2 = b_enc.reshape(1, H)
    zz = jnp.zeros((H, H), jnp.float32)
    Wg = jnp.concatenate([
        jnp.concatenate([W_ih[:2 * H], W_hh[:2 * H]], axis=1),
        jnp.concatenate([W_ih[2 * H:], zz], axis=1),
        jnp.concatenate([zz, W_hh[2 * H:]], axis=1),
    ], axis=0)
    bg = jnp.concatenate([b_ih[:2 * H] + b_hh[:2 * H], b_ih[2 * H:],
                          b_hh[2 * H:]]).reshape(1, 4 * H)
    Wm2 = jnp.concatenate([W_msg, W_msg], axis=0)
    bm2 = jnp.concatenate([b_msg, b_msg]).reshape(1, 2 * H)

    full = lambda shape: pl.BlockSpec(shape, lambda i: (0, 0))
    rows = lambda w: pl.BlockSpec((blk, w), lambda i: (i, 0))

    h, hw = pl.pallas_call(
        _enc_body,
        grid=(grid,),
        in_specs=[rows(d_feat), full((H, d_feat)), full((1, H)),
                  full((2 * H, H)), full((1, 2 * H))],
        out_specs=[rows(H), rows(2 * H)],
        out_shape=[jax.ShapeDtypeStruct((n_nodes, H), jnp.float32),
                   jax.ShapeDtypeStruct((n_nodes, 2 * H), jnp.float32)],
    )(x, W_enc, b_enc2, Wm2, bm2)

    p_specs = [pl.BlockSpec((blk, 2 * H), lambda i: (i, 0)),
               pl.BlockSpec((blk, 2 * H), lambda i: (i + grid, 0))]
    gru_call = pl.pallas_call(
        _gru_body,
        grid=(grid,),
        in_specs=p_specs + [rows(H), full((4 * H, 2 * H)), full((1, 4 * H)),
                            full((2 * H, H)), full((1, 2 * H))],
        out_specs=[rows(H), rows(2 * H)],
        out_shape=[jax.ShapeDtypeStruct((n_nodes, H), jnp.float32),
                   jax.ShapeDtypeStruct((n_nodes, 2 * H), jnp.float32)],
    )

    for _ in range(N_STEPS - 1):
        parts = sc_scatter(hw.reshape(2 * n_nodes, H), srcr, dstr)
        h, hw = gru_call(parts, parts, h, Wg, bg, Wm2, bm2)

    parts = sc_scatter(hw.reshape(2 * n_nodes, H), srcr, dstr)
    out = pl.pallas_call(
        _gru_final_body,
        grid=(grid,),
        in_specs=p_specs + [rows(H), full((4 * H, 2 * H)), full((1, 4 * H)),
                            full((H, H)), full((1, H)),
                            full((out_dim, H)), full((1, out_dim))],
        out_specs=pl.BlockSpec((1, out_dim), lambda i: (0, 0)),
        out_shape=jax.ShapeDtypeStruct((1, out_dim), jnp.float32),
        scratch_shapes=[pltpu.VMEM((1, H), jnp.float32)],
    )(parts, parts, h, Wg, bg, W_r1, b_r1.reshape(1, H),
      W_r2, b_r2.reshape(1, out_dim))
    return out
